# baseline (speedup 1.0000x reference)
import jax
import jax.numpy as jnp
from jax.experimental import pallas as pl

N_NODES = 10000
TEMP = 0.5


def _graph_conv(feat, src, dst, W, b, n_nodes):
    x = feat @ W
    deg_out = jnp.maximum(jnp.bincount(src, length=n_nodes), 1).astype(x.dtype)
    deg_in = jnp.maximum(jnp.bincount(dst, length=n_nodes), 1).astype(x.dtype)
    norm_src = deg_out ** -0.5
    norm_dst = deg_in ** -0.5
    msg = x[src] * norm_src[src][:, None]
    agg = jax.ops.segment_sum(msg, dst, num_segments=n_nodes)
    return agg * norm_dst[:, None] + b


def _encoder(feat, src, dst, W1, b1, W2, b2, n_nodes):
    h = jax.nn.relu(_graph_conv(feat, src, dst, W1, b1, n_nodes))
    h = jax.nn.relu(_graph_conv(h, src, dst, W2, b2, n_nodes))
    return h


def _proj(x, fc1_W, fc1_b, fc2_W, fc2_b):
    z = jax.nn.elu(x @ fc1_W + fc1_b)
    return z @ fc2_W + fc2_b


def _normalize(z, eps=1e-12):
    n = jnp.maximum(jnp.linalg.norm(z, axis=1, keepdims=True), eps)
    return z / n


def _get_loss(z1, z2, temp):
    z1n = _normalize(z1)
    z2n = _normalize(z2)
    refl_sim = jnp.exp((z1n @ z1n.T) / temp)
    between_sim = jnp.exp((z1n @ z2n.T) / temp)
    x1 = refl_sim.sum(1) + between_sim.sum(1) - jnp.diag(refl_sim)
    return -jnp.log(jnp.diag(between_sim) / x1)


def kernel(graph1, graph2, feat1, feat2, W1, b1, W2, b2, fc1_W, fc1_b, fc2_W, fc2_b):
    src1, dst1 = graph1[0], graph1[1]
    src2, dst2 = graph2[0], graph2[1]
    h1 = _encoder(feat1, src1, dst1, W1, b1, W2, b2, N_NODES)
    h2 = _encoder(feat2, src2, dst2, W1, b1, W2, b2, N_NODES)
    z1 = _proj(h1, fc1_W, fc1_b, fc2_W, fc2_b)
    z2 = _proj(h2, fc1_W, fc1_b, fc2_W, fc2_b)
    l1 = _get_loss(z1, z2, TEMP)
    l2 = _get_loss(z2, z1, TEMP)
    ret = (l1 + l2) * 0.5
    return ret.mean()


# speedup vs baseline: 5.2566x; 5.2566x over previous
import functools

import jax
import jax.numpy as jnp
import numpy as np
from jax import lax
from jax.experimental import pallas as pl
from jax.experimental.pallas import tpu as pltpu
import jax.experimental.pallas.tpu_sc as plsc

N = 10000
NPAD = 10240
E = 320000
D = 128
H2 = 256
TEMP = 0.5
INV_T = 1.0 / TEMP
E2 = float(np.exp(1.0 / TEMP))

NT = 16
EDG_T = E // NT
CH = 80
NIT = EDG_T // CH
RPT = NPAD // NT

def _deg_body(g_hbm, zeros1_hbm, out_hbm, sidx, didx, ones, acc_s, acc_d):
    c = lax.axis_index("c")
    s = lax.axis_index("s")
    for k in range(CH // 16):
        ones[pl.ds(k * 16, 16)] = jnp.ones((16,), jnp.float32)
    pltpu.sync_copy(zeros1_hbm.at[pl.ds(s * RPT, RPT)], acc_s.at[pl.ds(s * RPT, RPT)])
    pltpu.sync_copy(zeros1_hbm.at[pl.ds(s * RPT, RPT)], acc_d.at[pl.ds(s * RPT, RPT)])
    plsc.subcore_barrier()

    base = s * EDG_T
    sbase = c * (2 * E) + base
    dbase = c * (2 * E) + E + base

    def body(t, carry):
        off = t * CH
        pltpu.sync_copy(g_hbm.at[pl.ds(sbase + off, CH)], sidx)
        pltpu.sync_copy(g_hbm.at[pl.ds(dbase + off, CH)], didx)
        pltpu.sync_copy(ones, acc_s.at[sidx], add=True)
        pltpu.sync_copy(ones, acc_d.at[didx], add=True)
        return carry

    lax.fori_loop(0, NIT, body, 0)
    plsc.subcore_barrier()
    sl = pl.ds(s * RPT, RPT)
    pltpu.sync_copy(acc_s.at[sl], out_hbm.at[pl.ds(c * 2 * NPAD + s * RPT, RPT)])
    pltpu.sync_copy(acc_d.at[sl], out_hbm.at[pl.ds((c * 2 + 1) * NPAD + s * RPT, RPT)])


@functools.cache
def _make_deg_kernel():
    mesh = plsc.VectorSubcoreMesh(core_axis_name="c", subcore_axis_name="s",
                                  num_cores=2, num_subcores=NT)
    return pl.kernel(
        _deg_body,
        out_type=jax.ShapeDtypeStruct((4 * NPAD,), jnp.float32),
        mesh=mesh,
        scratch_types=[
            pltpu.VMEM((CH,), jnp.int32),
            pltpu.VMEM((CH,), jnp.int32),
            pltpu.VMEM((CH,), jnp.float32),
            pltpu.VMEM_SHARED((NPAD,), jnp.float32),
            pltpu.VMEM_SHARED((NPAD,), jnp.float32),
        ],
    )


def _deg_kernel(g_all, zeros1):
    return _make_deg_kernel()(g_all, zeros1)


def _segsum_body(xs_hbm, g_hbm, zeros2_hbm, out_hbm, sidx, sadj, didx, rows,
                 acc, sem):
    c = lax.axis_index("c")
    s = lax.axis_index("s")
    pltpu.sync_copy(zeros2_hbm.at[pl.ds(s * RPT, RPT)], acc.at[pl.ds(s * RPT, RPT)])
    plsc.subcore_barrier()

    base = s * EDG_T
    sbase = c * (2 * E) + base
    dbase = c * (2 * E) + E + base
    row_off = c * NPAD

    def body(t, carry):
        off = t * CH
        pltpu.sync_copy(g_hbm.at[pl.ds(sbase + off, CH)], sidx)
        pltpu.sync_copy(g_hbm.at[pl.ds(dbase + off, CH)], didx)
        for k in range(CH // 16):
            sadj[pl.ds(k * 16, 16)] = sidx[pl.ds(k * 16, 16)] + row_off
        pltpu.async_copy(xs_hbm.at[sadj], rows, sem).wait()
        pltpu.sync_copy(rows, acc.at[didx], add=True)
        return carry

    lax.fori_loop(0, NIT, body, 0)
    plsc.subcore_barrier()
    sl = pl.ds(s * RPT, RPT)
    pltpu.sync_copy(acc.at[sl], out_hbm.at[c, sl])


@functools.cache
def _make_segsum_kernel():
    mesh = plsc.VectorSubcoreMesh(core_axis_name="c", subcore_axis_name="s",
                                  num_cores=2, num_subcores=NT)
    return pl.kernel(
        _segsum_body,
        out_type=jax.ShapeDtypeStruct((2, NPAD, D), jnp.float32),
        mesh=mesh,
        scratch_types=[
            pltpu.VMEM((CH,), jnp.int32),
            pltpu.VMEM((CH,), jnp.int32),
            pltpu.VMEM((CH,), jnp.int32),
            pltpu.VMEM((CH, D), jnp.float32),
            pltpu.VMEM_SHARED((NPAD, D), jnp.float32),
            pltpu.SemaphoreType.DMA,
        ],
    )


def _segsum_kernel(xs_flat, g_all, zeros2):
    return _make_segsum_kernel()(xs_flat, g_all, zeros2)


def _prescale_body(feat_ref, deg_ref, xs_ref, ns_ref, nd_ref):
    f = feat_ref[0]
    dsrc = deg_ref[0, 0]
    ddst = deg_ref[0, 1]
    ns = lax.rsqrt(jnp.maximum(dsrc, 1.0))
    nd = lax.rsqrt(jnp.maximum(ddst, 1.0))
    ns_ref[0] = ns
    nd_ref[0] = nd
    xs_ref[0] = f * ns


def _prescale_call(feats, degs4):
    rb = 1024
    nb = NPAD // rb
    return pl.pallas_call(
        _prescale_body,
        grid=(2, nb),
        in_specs=[
            pl.BlockSpec((1, rb, D), lambda g, i: (g, i, 0)),
            pl.BlockSpec((1, 2, rb, 1), lambda g, i: (g, 0, i, 0)),
        ],
        out_specs=[
            pl.BlockSpec((1, rb, D), lambda g, i: (g, i, 0)),
            pl.BlockSpec((1, rb, 1), lambda g, i: (g, i, 0)),
            pl.BlockSpec((1, rb, 1), lambda g, i: (g, i, 0)),
        ],
        out_shape=[
            jax.ShapeDtypeStruct((2, NPAD, D), jnp.float32),
            jax.ShapeDtypeStruct((2, NPAD, 1), jnp.float32),
            jax.ShapeDtypeStruct((2, NPAD, 1), jnp.float32),
        ],
    )(feats, degs4)


_RB = 512
_NB = NPAD // _RB


def _mlp1_body(agg_ref, ns_ref, nd_ref, w1_ref, b1_ref, w2_ref, ys_ref):
    a = agg_ref[0]
    h = jnp.dot(a, w1_ref[...], preferred_element_type=jnp.float32)
    h = jnp.maximum(h * nd_ref[0] + b1_ref[...], 0.0)
    y = jnp.dot(h, w2_ref[...], preferred_element_type=jnp.float32)
    ys_ref[0] = y * ns_ref[0]


def _mlp1_call(agg, ns, nd, W1, b1, W2):
    return pl.pallas_call(
        _mlp1_body,
        grid=(2, _NB),
        in_specs=[
            pl.BlockSpec((1, _RB, D), lambda g, i: (g, i, 0)),
            pl.BlockSpec((1, _RB, 1), lambda g, i: (g, i, 0)),
            pl.BlockSpec((1, _RB, 1), lambda g, i: (g, i, 0)),
            pl.BlockSpec((D, H2), lambda g, i: (0, 0)),
            pl.BlockSpec((1, H2), lambda g, i: (0, 0)),
            pl.BlockSpec((H2, D), lambda g, i: (0, 0)),
        ],
        out_specs=pl.BlockSpec((1, _RB, D), lambda g, i: (g, i, 0)),
        out_shape=jax.ShapeDtypeStruct((2, NPAD, D), jnp.float32),
    )(agg, ns, nd, W1, b1, W2)


def _proj_body(agg_ref, nd_ref, b2_ref, f1w_ref, f1b_ref, f2w_ref, f2b_ref,
               zn_ref):
    i = pl.program_id(1)
    h = jnp.maximum(agg_ref[0] * nd_ref[0] + b2_ref[...], 0.0)
    t = jnp.dot(h, f1w_ref[...], preferred_element_type=jnp.float32) + f1b_ref[...]
    e = jnp.where(t > 0.0, t, jnp.exp(t) - 1.0)
    z = jnp.dot(e, f2w_ref[...], preferred_element_type=jnp.float32) + f2b_ref[...]
    nrm = jnp.sqrt(jnp.sum(z * z, axis=1, keepdims=True))
    zn = z / jnp.maximum(nrm, 1e-12)
    rows = lax.broadcasted_iota(jnp.int32, (_RB, 1), 0) + i * _RB
    zn = jnp.where(rows < N, zn, 0.0)
    zn_ref[0] = zn.astype(jnp.bfloat16)


def _proj_call(agg2, nd, b2, fc1_W, fc1_b, fc2_W, fc2_b):
    return pl.pallas_call(
        _proj_body,
        grid=(2, _NB),
        in_specs=[
            pl.BlockSpec((1, _RB, D), lambda g, i: (g, i, 0)),
            pl.BlockSpec((1, _RB, 1), lambda g, i: (g, i, 0)),
            pl.BlockSpec((1, D), lambda g, i: (0, 0)),
            pl.BlockSpec((D, D), lambda g, i: (0, 0)),
            pl.BlockSpec((1, D), lambda g, i: (0, 0)),
            pl.BlockSpec((D, D), lambda g, i: (0, 0)),
            pl.BlockSpec((1, D), lambda g, i: (0, 0)),
        ],
        out_specs=pl.BlockSpec((1, _RB, D), lambda g, i: (g, i, 0)),
        out_shape=jax.ShapeDtypeStruct((2, NPAD, D), jnp.bfloat16),
    )(agg2, nd, b2, fc1_W, fc1_b, fc2_W, fc2_b)


_RI = 512
_CJ = 2048
_NBI = NPAD // _RI
_NBJ = NPAD // _CJ
_DN = (((1,), (1,)), ((), ()))


def _loss_body(zr_ref, zc_ref, out_ref, rsA, rsB, rsC, rsD, dg):
    i = pl.program_id(0)
    j = pl.program_id(1)
    z1r = zr_ref[0]
    z2r = zr_ref[1]
    z1c = zc_ref[0]
    z2c = zc_ref[1]
    sa_m = lax.dot_general(z1r, z1c, _DN, preferred_element_type=jnp.float32)
    sb_m = lax.dot_general(z2r, z2c, _DN, preferred_element_type=jnp.float32)
    sc_m = lax.dot_general(z1r, z2c, _DN, preferred_element_type=jnp.float32)
    sd_m = lax.dot_general(z2r, z1c, _DN, preferred_element_type=jnp.float32)
    colg = lax.broadcasted_iota(jnp.int32, (_RI, _CJ), 1) + j * _CJ
    cval = colg < N
    zero = jnp.float32(0.0)
    ea = jnp.where(cval, jnp.exp(INV_T * sa_m), zero)
    eb = jnp.where(cval, jnp.exp(INV_T * sb_m), zero)
    ec = jnp.where(cval, jnp.exp(INV_T * sc_m), zero)
    ed = jnp.where(cval, jnp.exp(INV_T * sd_m), zero)
    sa = jnp.sum(ea, axis=1, keepdims=True)
    sb = jnp.sum(eb, axis=1, keepdims=True)
    sc = jnp.sum(ec, axis=1, keepdims=True)
    sd = jnp.sum(ed, axis=1, keepdims=True)
    rowg = lax.broadcasted_iota(jnp.int32, (_RI, _CJ), 0) + i * _RI
    dd = jnp.sum(jnp.where(rowg == colg, INV_T * sc_m, zero), axis=1,
                 keepdims=True)
    sl = pl.ds(i * _RI, _RI)

    @pl.when(j == 0)
    def _():
        rsA[sl, :] = sa
        rsB[sl, :] = sb
        rsC[sl, :] = sc
        rsD[sl, :] = sd
        dg[sl, :] = dd

    @pl.when(j > 0)
    def _():
        rsA[sl, :] += sa
        rsB[sl, :] += sb
        rsC[sl, :] += sc
        rsD[sl, :] += sd
        dg[sl, :] += dd

    @pl.when((i == _NBI - 1) & (j == _NBJ - 1))
    def _():
        x1 = rsA[...] + rsC[...] - E2
        x2 = rsB[...] + rsD[...] - E2
        lv = -dg[...] + 0.5 * (jnp.log(x1) + jnp.log(x2))
        rows = lax.broadcasted_iota(jnp.int32, (NPAD, 1), 0)
        tot = jnp.sum(jnp.where(rows < N, lv, zero), keepdims=True)
        out_ref[...] = tot / N


def _loss_call(zn):
    return pl.pallas_call(
        _loss_body,
        grid=(_NBI, _NBJ),
        in_specs=[
            pl.BlockSpec((2, _RI, D), lambda i, j: (0, i, 0)),
            pl.BlockSpec((2, _CJ, D), lambda i, j: (0, j, 0)),
        ],
        out_specs=pl.BlockSpec((1, 1), lambda i, j: (0, 0)),
        out_shape=jax.ShapeDtypeStruct((1, 1), jnp.float32),
        scratch_shapes=[pltpu.VMEM((NPAD, 1), jnp.float32) for _ in range(5)],
    )(zn, zn)


def kernel(graph1, graph2, feat1, feat2, W1, b1, W2, b2, fc1_W, fc1_b, fc2_W,
           fc2_b):
    g_flat = jnp.concatenate([graph1.astype(jnp.int32).reshape(-1),
                              graph2.astype(jnp.int32).reshape(-1)])
    f1p = jnp.pad(feat1, ((0, NPAD - N), (0, 0)))
    f2p = jnp.pad(feat2, ((0, NPAD - N), (0, 0)))
    feats = jnp.stack([f1p, f2p])
    zeros1 = jnp.zeros((NPAD,), jnp.float32)
    zeros2 = jnp.zeros((NPAD, D), jnp.float32)

    degs = _deg_kernel(g_flat, zeros1)
    xs, ns, nd = _prescale_call(feats, degs.reshape(2, 2, NPAD, 1))
    agg = _segsum_kernel(xs.reshape(2 * NPAD, D), g_flat, zeros2)
    ys = _mlp1_call(agg, ns, nd, W1, b1.reshape(1, H2), W2)
    agg2 = _segsum_kernel(ys.reshape(2 * NPAD, D), g_flat, zeros2)
    zn = _proj_call(agg2, nd, b2.reshape(1, D), fc1_W, fc1_b.reshape(1, D),
                    fc2_W, fc2_b.reshape(1, D))
    out = _loss_call(zn)
    return out.reshape(())


# speedup vs baseline: 8.0212x; 1.5259x over previous
import functools

import jax
import jax.numpy as jnp
import numpy as np
from jax import lax
from jax.experimental import pallas as pl
from jax.experimental.pallas import tpu as pltpu
import jax.experimental.pallas.tpu_sc as plsc

N = 10000
NPAD = 10240
E = 320000
D = 128
H2 = 256
TEMP = 0.5
INV_T = 1.0 / TEMP
E2 = float(np.exp(1.0 / TEMP))

NT = 16
EDG_T = E // NT
CH = 80
NIT = EDG_T // CH
RPT = NPAD // NT
EROWS = E // CH
G_ROWS = 4 * EROWS

_DGRP = 10


def _deg_body(g_hbm, zeros1_hbm, ones_hbm, out_hbm, sidx0, didx0, sidx1,
              didx1, ones, acc_s, acc_d, isem0, isem1, ss0, ss1):
    c = lax.axis_index("c")
    s = lax.axis_index("s")
    pltpu.sync_copy(ones_hbm, ones)
    pltpu.sync_copy(zeros1_hbm.at[pl.ds(s * RPT, RPT)], acc_s.at[pl.ds(s * RPT, RPT)])
    pltpu.sync_copy(zeros1_hbm.at[pl.ds(s * RPT, RPT)], acc_d.at[pl.ds(s * RPT, RPT)])
    plsc.subcore_barrier()

    sbase = c * (2 * E) + s * EDG_T
    dbase = sbase + E

    def idx_start(t, sbuf, dbuf, sem):
        pltpu.async_copy(g_hbm.at[pl.ds(sbase + t * CH, CH)], sbuf, sem)
        pltpu.async_copy(g_hbm.at[pl.ds(dbase + t * CH, CH)], dbuf, sem)

    def idx_wait(t, sbuf, dbuf, sem):
        pltpu.make_async_copy(g_hbm.at[pl.ds(sbase + t * CH, CH)], sbuf, sem).wait()
        pltpu.make_async_copy(g_hbm.at[pl.ds(dbase + t * CH, CH)], dbuf, sem).wait()

    def sc_start(sbuf, dbuf, sem):
        pltpu.async_copy(ones, acc_s.at[sbuf], sem, add=True)
        pltpu.async_copy(ones, acc_d.at[dbuf], sem, add=True)

    def sc_wait(sbuf, dbuf, sem):
        pltpu.make_async_copy(ones, acc_s.at[sbuf], sem).wait()
        pltpu.make_async_copy(ones, acc_d.at[dbuf], sem).wait()

    idx_start(0, sidx0, didx0, isem0)
    idx_start(1, sidx1, didx1, isem1)

    def body(g, carry):
        t0 = 2 * g
        idx_wait(t0, sidx0, didx0, isem0)
        sc_start(sidx0, didx0, ss0)
        idx_wait(t0 + 1, sidx1, didx1, isem1)
        sc_start(sidx1, didx1, ss1)

        @pl.when(g < NIT // 2 - 1)
        def _():
            sc_wait(sidx0, didx0, ss0)
            idx_start(t0 + 2, sidx0, didx0, isem0)
            sc_wait(sidx1, didx1, ss1)
            idx_start(t0 + 3, sidx1, didx1, isem1)

        return carry

    lax.fori_loop(0, NIT // 2, body, 0)
    sc_wait(sidx0, didx0, ss0)
    sc_wait(sidx1, didx1, ss1)
    plsc.subcore_barrier()
    sl = pl.ds(s * RPT, RPT)
    pltpu.sync_copy(acc_s.at[sl], out_hbm.at[pl.ds(c * 2 * NPAD + s * RPT, RPT)])
    pltpu.sync_copy(acc_d.at[sl], out_hbm.at[pl.ds((c * 2 + 1) * NPAD + s * RPT, RPT)])


@functools.cache
def _make_deg_kernel():
    mesh = plsc.VectorSubcoreMesh(core_axis_name="c", subcore_axis_name="s",
                                  num_cores=2, num_subcores=NT)
    return pl.kernel(
        _deg_body,
        out_type=jax.ShapeDtypeStruct((4 * NPAD,), jnp.float32),
        mesh=mesh,
        scratch_types=[
            pltpu.VMEM((CH,), jnp.int32),
            pltpu.VMEM((CH,), jnp.int32),
            pltpu.VMEM((CH,), jnp.int32),
            pltpu.VMEM((CH,), jnp.int32),
            pltpu.VMEM((CH,), jnp.float32),
            pltpu.VMEM_SHARED((NPAD,), jnp.float32),
            pltpu.VMEM_SHARED((NPAD,), jnp.float32),
            pltpu.SemaphoreType.DMA,
            pltpu.SemaphoreType.DMA,
            pltpu.SemaphoreType.DMA,
            pltpu.SemaphoreType.DMA,
        ],
    )


def _deg_kernel(g_flat, zeros1, ones1):
    return _make_deg_kernel()(g_flat, zeros1, ones1)


def _segsum_body(xs_hbm, g_hbm, zeros2_hbm, out_hbm, sidx0, didx0, sidx1,
                 didx1, rows0, rows1, acc, isem0, isem1, gsem0, gsem1, ssem0,
                 ssem1):
    c = lax.axis_index("c")
    s = lax.axis_index("s")
    pltpu.sync_copy(zeros2_hbm.at[pl.ds(s * RPT, RPT)], acc.at[pl.ds(s * RPT, RPT)])
    plsc.subcore_barrier()

    sbase = c * (2 * E) + s * EDG_T
    dbase = sbase + E

    def idx_start(t, sbuf, dbuf, sem):
        pltpu.async_copy(g_hbm.at[pl.ds(sbase + t * CH, CH)], sbuf, sem)
        pltpu.async_copy(g_hbm.at[pl.ds(dbase + t * CH, CH)], dbuf, sem)

    def idx_wait(t, sbuf, dbuf, sem):
        pltpu.make_async_copy(g_hbm.at[pl.ds(sbase + t * CH, CH)], sbuf, sem).wait()
        pltpu.make_async_copy(g_hbm.at[pl.ds(dbase + t * CH, CH)], dbuf, sem).wait()

    idx_start(0, sidx0, didx0, isem0)
    idx_start(1, sidx1, didx1, isem1)

    def body(g, carry):
        t0 = 2 * g
        idx_wait(t0, sidx0, didx0, isem0)
        pltpu.async_copy(xs_hbm.at[sidx0], rows0, gsem0)
        idx_wait(t0 + 1, sidx1, didx1, isem1)
        pltpu.async_copy(xs_hbm.at[sidx1], rows1, gsem1)
        pltpu.make_async_copy(xs_hbm.at[sidx0], rows0, gsem0).wait()
        pltpu.async_copy(rows0, acc.at[didx0], ssem0, add=True)
        pltpu.make_async_copy(xs_hbm.at[sidx1], rows1, gsem1).wait()
        pltpu.async_copy(rows1, acc.at[didx1], ssem1, add=True)

        @pl.when(g < NIT // 2 - 1)
        def _():
            pltpu.make_async_copy(rows0, acc.at[didx0], ssem0).wait()
            idx_start(t0 + 2, sidx0, didx0, isem0)
            pltpu.make_async_copy(rows1, acc.at[didx1], ssem1).wait()
            idx_start(t0 + 3, sidx1, didx1, isem1)

        return carry

    lax.fori_loop(0, NIT // 2, body, 0)
    pltpu.make_async_copy(rows0, acc.at[didx0], ssem0).wait()
    pltpu.make_async_copy(rows1, acc.at[didx1], ssem1).wait()
    plsc.subcore_barrier()
    sl = pl.ds(s * RPT, RPT)
    pltpu.sync_copy(acc.at[sl], out_hbm.at[c, sl])


@functools.cache
def _make_segsum_kernel():
    mesh = plsc.VectorSubcoreMesh(core_axis_name="c", subcore_axis_name="s",
                                  num_cores=2, num_subcores=NT)
    return pl.kernel(
        _segsum_body,
        out_type=jax.ShapeDtypeStruct((2, NPAD, D), jnp.float32),
        mesh=mesh,
        scratch_types=[
            pltpu.VMEM((CH,), jnp.int32),
            pltpu.VMEM((CH,), jnp.int32),
            pltpu.VMEM((CH,), jnp.int32),
            pltpu.VMEM((CH,), jnp.int32),
            pltpu.VMEM((CH, D), jnp.float32),
            pltpu.VMEM((CH, D), jnp.float32),
            pltpu.VMEM_SHARED((NPAD, D), jnp.float32),
            pltpu.SemaphoreType.DMA,
            pltpu.SemaphoreType.DMA,
            pltpu.SemaphoreType.DMA,
            pltpu.SemaphoreType.DMA,
            pltpu.SemaphoreType.DMA,
            pltpu.SemaphoreType.DMA,
        ],
    )


def _segsum_kernel(xs_flat, g_adj_flat, zeros2):
    return _make_segsum_kernel()(xs_flat, g_adj_flat, zeros2)


def _prescale_body(feat_ref, deg_ref, g2d_ref, xs_ref, ns_ref, nd_ref,
                   gadj_ref):
    g = pl.program_id(0)
    i = pl.program_id(1)
    f = feat_ref[0]
    dsrc = deg_ref[0, 0]
    ddst = deg_ref[0, 1]
    ns = lax.rsqrt(jnp.maximum(dsrc, 1.0))
    nd = lax.rsqrt(jnp.maximum(ddst, 1.0))
    ns_ref[0] = ns
    nd_ref[0] = nd
    xs_ref[0] = f * ns

    @pl.when((g == 0) & (i == 0))
    def _():
        r = lax.broadcasted_iota(jnp.int32, (G_ROWS, 1), 0)
        adj = jnp.where((r >= 2 * EROWS) & (r < 3 * EROWS), NPAD, 0)
        gadj_ref[...] = g2d_ref[...] + adj


def _prescale_call(feats, degs4, g2d):
    rb = 1024
    nb = NPAD // rb
    return pl.pallas_call(
        _prescale_body,
        grid=(2, nb),
        in_specs=[
            pl.BlockSpec((1, rb, D), lambda g, i: (g, i, 0)),
            pl.BlockSpec((1, 2, rb, 1), lambda g, i: (g, 0, i, 0)),
            pl.BlockSpec((G_ROWS, CH), lambda g, i: (0, 0)),
        ],
        out_specs=[
            pl.BlockSpec((1, rb, D), lambda g, i: (g, i, 0)),
            pl.BlockSpec((1, rb, 1), lambda g, i: (g, i, 0)),
            pl.BlockSpec((1, rb, 1), lambda g, i: (g, i, 0)),
            pl.BlockSpec((G_ROWS, CH), lambda g, i: (0, 0)),
        ],
        out_shape=[
            jax.ShapeDtypeStruct((2, NPAD, D), jnp.float32),
            jax.ShapeDtypeStruct((2, NPAD, 1), jnp.float32),
            jax.ShapeDtypeStruct((2, NPAD, 1), jnp.float32),
            jax.ShapeDtypeStruct((G_ROWS, CH), jnp.int32),
        ],
    )(feats, degs4, g2d)


_RB = 512
_NB = NPAD // _RB


def _mlp1_body(agg_ref, ns_ref, nd_ref, w1_ref, b1_ref, w2_ref, ys_ref):
    a = agg_ref[0]
    h = jnp.dot(a, w1_ref[...], preferred_element_type=jnp.float32)
    h = jnp.maximum(h * nd_ref[0] + b1_ref[...], 0.0)
    y = jnp.dot(h, w2_ref[...], preferred_element_type=jnp.float32)
    ys_ref[0] = y * ns_ref[0]


def _mlp1_call(agg, ns, nd, W1, b1, W2):
    return pl.pallas_call(
        _mlp1_body,
        grid=(2, _NB),
        in_specs=[
            pl.BlockSpec((1, _RB, D), lambda g, i: (g, i, 0)),
            pl.BlockSpec((1, _RB, 1), lambda g, i: (g, i, 0)),
            pl.BlockSpec((1, _RB, 1), lambda g, i: (g, i, 0)),
            pl.BlockSpec((D, H2), lambda g, i: (0, 0)),
            pl.BlockSpec((1, H2), lambda g, i: (0, 0)),
            pl.BlockSpec((H2, D), lambda g, i: (0, 0)),
        ],
        out_specs=pl.BlockSpec((1, _RB, D), lambda g, i: (g, i, 0)),
        out_shape=jax.ShapeDtypeStruct((2, NPAD, D), jnp.float32),
    )(agg, ns, nd, W1, b1, W2)


def _proj_body(agg_ref, nd_ref, b2_ref, f1w_ref, f1b_ref, f2w_ref, f2b_ref,
               zn_ref):
    i = pl.program_id(1)
    h = jnp.maximum(agg_ref[0] * nd_ref[0] + b2_ref[...], 0.0)
    t = jnp.dot(h, f1w_ref[...], preferred_element_type=jnp.float32) + f1b_ref[...]
    e = jnp.where(t > 0.0, t, jnp.exp(t) - 1.0)
    z = jnp.dot(e, f2w_ref[...], preferred_element_type=jnp.float32) + f2b_ref[...]
    nrm = jnp.sqrt(jnp.sum(z * z, axis=1, keepdims=True))
    zn = z / jnp.maximum(nrm, 1e-12)
    rows = lax.broadcasted_iota(jnp.int32, (_RB, 1), 0) + i * _RB
    zn = jnp.where(rows < N, zn, 0.0)
    zn_ref[0] = zn.astype(jnp.bfloat16)


def _proj_call(agg2, nd, b2, fc1_W, fc1_b, fc2_W, fc2_b):
    return pl.pallas_call(
        _proj_body,
        grid=(2, _NB),
        in_specs=[
            pl.BlockSpec((1, _RB, D), lambda g, i: (g, i, 0)),
            pl.BlockSpec((1, _RB, 1), lambda g, i: (g, i, 0)),
            pl.BlockSpec((1, D), lambda g, i: (0, 0)),
            pl.BlockSpec((D, D), lambda g, i: (0, 0)),
            pl.BlockSpec((1, D), lambda g, i: (0, 0)),
            pl.BlockSpec((D, D), lambda g, i: (0, 0)),
            pl.BlockSpec((1, D), lambda g, i: (0, 0)),
        ],
        out_specs=pl.BlockSpec((1, _RB, D), lambda g, i: (g, i, 0)),
        out_shape=jax.ShapeDtypeStruct((2, NPAD, D), jnp.bfloat16),
    )(agg2, nd, b2, fc1_W, fc1_b, fc2_W, fc2_b)


_RI = 512
_CJ = 2048
_NBI = NPAD // _RI
_NBJ = NPAD // _CJ
_DN = (((1,), (1,)), ((), ()))


def _loss_body(zr_ref, zc_ref, out_ref, rsA, rsB, rsC, rsD, dg):
    i = pl.program_id(0)
    j = pl.program_id(1)
    z1r = zr_ref[0]
    z2r = zr_ref[1]
    z1c = zc_ref[0]
    z2c = zc_ref[1]
    sa_m = lax.dot_general(z1r, z1c, _DN, preferred_element_type=jnp.float32)
    sb_m = lax.dot_general(z2r, z2c, _DN, preferred_element_type=jnp.float32)
    sc_m = lax.dot_general(z1r, z2c, _DN, preferred_element_type=jnp.float32)
    sd_m = lax.dot_general(z2r, z1c, _DN, preferred_element_type=jnp.float32)
    colg = lax.broadcasted_iota(jnp.int32, (_RI, _CJ), 1) + j * _CJ
    cval = colg < N
    zero = jnp.float32(0.0)
    ea = jnp.where(cval, jnp.exp(INV_T * sa_m), zero)
    eb = jnp.where(cval, jnp.exp(INV_T * sb_m), zero)
    ec = jnp.where(cval, jnp.exp(INV_T * sc_m), zero)
    ed = jnp.where(cval, jnp.exp(INV_T * sd_m), zero)
    sa = jnp.sum(ea, axis=1, keepdims=True)
    sb = jnp.sum(eb, axis=1, keepdims=True)
    sc = jnp.sum(ec, axis=1, keepdims=True)
    sd = jnp.sum(ed, axis=1, keepdims=True)
    rowg = lax.broadcasted_iota(jnp.int32, (_RI, _CJ), 0) + i * _RI
    dd = jnp.sum(jnp.where(rowg == colg, INV_T * sc_m, zero), axis=1,
                 keepdims=True)
    sl = pl.ds(i * _RI, _RI)

    @pl.when(j == 0)
    def _():
        rsA[sl, :] = sa
        rsB[sl, :] = sb
        rsC[sl, :] = sc
        rsD[sl, :] = sd
        dg[sl, :] = dd

    @pl.when(j > 0)
    def _():
        rsA[sl, :] += sa
        rsB[sl, :] += sb
        rsC[sl, :] += sc
        rsD[sl, :] += sd
        dg[sl, :] += dd

    @pl.when((i == _NBI - 1) & (j == _NBJ - 1))
    def _():
        x1 = rsA[...] + rsC[...] - E2
        x2 = rsB[...] + rsD[...] - E2
        lv = -dg[...] + 0.5 * (jnp.log(x1) + jnp.log(x2))
        rows = lax.broadcasted_iota(jnp.int32, (NPAD, 1), 0)
        tot = jnp.sum(jnp.where(rows < N, lv, zero), keepdims=True)
        out_ref[...] = tot / N


def _loss_call(zn):
    return pl.pallas_call(
        _loss_body,
        grid=(_NBI, _NBJ),
        in_specs=[
            pl.BlockSpec((2, _RI, D), lambda i, j: (0, i, 0)),
            pl.BlockSpec((2, _CJ, D), lambda i, j: (0, j, 0)),
        ],
        out_specs=pl.BlockSpec((1, 1), lambda i, j: (0, 0)),
        out_shape=jax.ShapeDtypeStruct((1, 1), jnp.float32),
        scratch_shapes=[pltpu.VMEM((NPAD, 1), jnp.float32) for _ in range(5)],
    )(zn, zn)


def kernel(graph1, graph2, feat1, feat2, W1, b1, W2, b2, fc1_W, fc1_b, fc2_W,
           fc2_b):
    g2d = jnp.concatenate([graph1.astype(jnp.int32).reshape(-1),
                           graph2.astype(jnp.int32).reshape(-1)]
                          ).reshape(G_ROWS, CH)
    f1p = jnp.pad(feat1, ((0, NPAD - N), (0, 0)))
    f2p = jnp.pad(feat2, ((0, NPAD - N), (0, 0)))
    feats = jnp.stack([f1p, f2p])
    zeros1 = jnp.zeros((NPAD,), jnp.float32)
    zeros2 = jnp.zeros((NPAD, D), jnp.float32)
    ones1 = jnp.ones((CH,), jnp.float32)

    degs = _deg_kernel(g2d.reshape(-1), zeros1, ones1)
    xs, ns, nd, g2d_adj = _prescale_call(feats, degs.reshape(2, 2, NPAD, 1),
                                         g2d)
    g_adj = g2d_adj.reshape(-1)
    agg = _segsum_kernel(xs.reshape(2 * NPAD, D), g_adj, zeros2)
    ys = _mlp1_call(agg, ns, nd, W1, b1.reshape(1, H2), W2)
    agg2 = _segsum_kernel(ys.reshape(2 * NPAD, D), g_adj, zeros2)
    zn = _proj_call(agg2, nd, b2.reshape(1, D), fc1_W, fc1_b.reshape(1, D),
                    fc2_W, fc2_b.reshape(1, D))
    out = _loss_call(zn)
    return out.reshape(())


# speedup vs baseline: 8.6490x; 1.0783x over previous
import functools

import jax
import jax.numpy as jnp
import numpy as np
from jax import lax
from jax.experimental import pallas as pl
from jax.experimental.pallas import tpu as pltpu
import jax.experimental.pallas.tpu_sc as plsc

N = 10000
NPAD = 10240
E = 320000
D = 128
H2 = 256
TEMP = 0.5
INV_T = 1.0 / TEMP
E2 = float(np.exp(1.0 / TEMP))

NT = 16
EDG_T = E // NT
CH = 40
NIT = EDG_T // CH
RPT = NPAD // NT
EROWS = E // CH
G_ROWS = 4 * EROWS

_DGRP = 10


def _deg_body(g_hbm, zeros1_hbm, ones_hbm, out_hbm, sidx0, didx0, sidx1,
              didx1, ones, acc_s, acc_d, isem0, isem1, ss0, ss1):
    c = lax.axis_index("c")
    s = lax.axis_index("s")
    pltpu.sync_copy(ones_hbm, ones)
    pltpu.sync_copy(zeros1_hbm.at[pl.ds(s * RPT, RPT)], acc_s.at[pl.ds(s * RPT, RPT)])
    pltpu.sync_copy(zeros1_hbm.at[pl.ds(s * RPT, RPT)], acc_d.at[pl.ds(s * RPT, RPT)])
    plsc.subcore_barrier()

    sbase = c * (2 * E) + s * EDG_T
    dbase = sbase + E

    def idx_start(t, sbuf, dbuf, sem):
        pltpu.async_copy(g_hbm.at[pl.ds(sbase + t * CH, CH)], sbuf, sem)
        pltpu.async_copy(g_hbm.at[pl.ds(dbase + t * CH, CH)], dbuf, sem)

    def idx_wait(t, sbuf, dbuf, sem):
        pltpu.make_async_copy(g_hbm.at[pl.ds(sbase + t * CH, CH)], sbuf, sem).wait()
        pltpu.make_async_copy(g_hbm.at[pl.ds(dbase + t * CH, CH)], dbuf, sem).wait()

    def sc_start(sbuf, dbuf, sem):
        pltpu.async_copy(ones, acc_s.at[sbuf], sem, add=True)
        pltpu.async_copy(ones, acc_d.at[dbuf], sem, add=True)

    def sc_wait(sbuf, dbuf, sem):
        pltpu.make_async_copy(ones, acc_s.at[sbuf], sem).wait()
        pltpu.make_async_copy(ones, acc_d.at[dbuf], sem).wait()

    idx_start(0, sidx0, didx0, isem0)
    idx_start(1, sidx1, didx1, isem1)

    def body(g, carry):
        t0 = 2 * g
        idx_wait(t0, sidx0, didx0, isem0)
        sc_start(sidx0, didx0, ss0)
        idx_wait(t0 + 1, sidx1, didx1, isem1)
        sc_start(sidx1, didx1, ss1)

        @pl.when(g < NIT // 2 - 1)
        def _():
            sc_wait(sidx0, didx0, ss0)
            idx_start(t0 + 2, sidx0, didx0, isem0)
            sc_wait(sidx1, didx1, ss1)
            idx_start(t0 + 3, sidx1, didx1, isem1)

        return carry

    lax.fori_loop(0, NIT // 2, body, 0)
    sc_wait(sidx0, didx0, ss0)
    sc_wait(sidx1, didx1, ss1)
    plsc.subcore_barrier()
    sl = pl.ds(s * RPT, RPT)
    pltpu.sync_copy(acc_s.at[sl], out_hbm.at[pl.ds(c * 2 * NPAD + s * RPT, RPT)])
    pltpu.sync_copy(acc_d.at[sl], out_hbm.at[pl.ds((c * 2 + 1) * NPAD + s * RPT, RPT)])


@functools.cache
def _make_deg_kernel():
    mesh = plsc.VectorSubcoreMesh(core_axis_name="c", subcore_axis_name="s",
                                  num_cores=2, num_subcores=NT)
    return pl.kernel(
        _deg_body,
        out_type=jax.ShapeDtypeStruct((4 * NPAD,), jnp.float32),
        mesh=mesh,
        scratch_types=[
            pltpu.VMEM((CH,), jnp.int32),
            pltpu.VMEM((CH,), jnp.int32),
            pltpu.VMEM((CH,), jnp.int32),
            pltpu.VMEM((CH,), jnp.int32),
            pltpu.VMEM((CH,), jnp.float32),
            pltpu.VMEM_SHARED((NPAD,), jnp.float32),
            pltpu.VMEM_SHARED((NPAD,), jnp.float32),
            pltpu.SemaphoreType.DMA,
            pltpu.SemaphoreType.DMA,
            pltpu.SemaphoreType.DMA,
            pltpu.SemaphoreType.DMA,
        ],
    )


def _deg_kernel(g_flat, zeros1, ones1):
    return _make_deg_kernel()(g_flat, zeros1, ones1)


_RING = 5


def _segsum_body(xs_hbm, g_hbm, zeros2_hbm, out_hbm, sidx, didx, rows, acc,
                 isem, gsem, ssem):
    c = lax.axis_index("c")
    s = lax.axis_index("s")
    pltpu.sync_copy(zeros2_hbm.at[pl.ds(s * RPT, RPT)], acc.at[pl.ds(s * RPT, RPT)])
    plsc.subcore_barrier()

    sbase = c * (2 * E) + s * EDG_T
    dbase = sbase + E

    def idx_start(t, b):
        pltpu.async_copy(g_hbm.at[pl.ds(sbase + t * CH, CH)], sidx[b], isem[b])
        pltpu.async_copy(g_hbm.at[pl.ds(dbase + t * CH, CH)], didx[b], isem[b])

    def idx_wait(t, b):
        pltpu.make_async_copy(g_hbm.at[pl.ds(sbase + t * CH, CH)], sidx[b], isem[b]).wait()
        pltpu.make_async_copy(g_hbm.at[pl.ds(dbase + t * CH, CH)], didx[b], isem[b]).wait()

    def gather_start(b):
        pltpu.async_copy(xs_hbm.at[sidx[b]], rows[b], gsem[b])

    def gather_wait(b):
        pltpu.make_async_copy(xs_hbm.at[sidx[b]], rows[b], gsem[b]).wait()

    def scatter_start(b):
        pltpu.async_copy(rows[b], acc.at[didx[b]], ssem[b], add=True)

    def scatter_wait(b):
        pltpu.make_async_copy(rows[b], acc.at[didx[b]], ssem[b]).wait()

    idx_start(0, 0)
    idx_start(1, 1)
    idx_wait(0, 0)
    gather_start(0)

    def body(g, carry):
        for b in range(_RING):
            t = _RING * g + b
            bi = (b + 2) % _RING

            @pl.when((t >= 3) & (t + 2 <= NIT - 1))
            def _():
                scatter_wait(bi)

            @pl.when(t + 2 <= NIT - 1)
            def _():
                idx_start(t + 2, bi)

            bg = (b + 1) % _RING

            @pl.when(t + 1 <= NIT - 1)
            def _():
                idx_wait(t + 1, bg)
                gather_start(bg)

            gather_wait(b)
            scatter_start(b)
        return carry

    lax.fori_loop(0, NIT // _RING, body, 0)
    for t in range(NIT - _RING, NIT):
        scatter_wait(t % _RING)
    plsc.subcore_barrier()
    sl = pl.ds(s * RPT, RPT)
    pltpu.sync_copy(acc.at[sl], out_hbm.at[c, sl])


@functools.cache
def _make_segsum_kernel():
    mesh = plsc.VectorSubcoreMesh(core_axis_name="c", subcore_axis_name="s",
                                  num_cores=2, num_subcores=NT)
    return pl.kernel(
        _segsum_body,
        out_type=jax.ShapeDtypeStruct((2, NPAD, D), jnp.float32),
        mesh=mesh,
        scratch_types=[
            [pltpu.VMEM((CH,), jnp.int32) for _ in range(_RING)],
            [pltpu.VMEM((CH,), jnp.int32) for _ in range(_RING)],
            [pltpu.VMEM((CH, D), jnp.float32) for _ in range(_RING)],
            pltpu.VMEM_SHARED((NPAD, D), jnp.float32),
            [pltpu.SemaphoreType.DMA for _ in range(_RING)],
            [pltpu.SemaphoreType.DMA for _ in range(_RING)],
            [pltpu.SemaphoreType.DMA for _ in range(_RING)],
        ],
    )


def _segsum_kernel(xs_flat, g_adj_flat, zeros2):
    return _make_segsum_kernel()(xs_flat, g_adj_flat, zeros2)


def _prescale_body(feat_ref, deg_ref, g2d_ref, xs_ref, ns_ref, nd_ref,
                   gadj_ref):
    g = pl.program_id(0)
    i = pl.program_id(1)
    f = feat_ref[0]
    dsrc = deg_ref[0, 0]
    ddst = deg_ref[0, 1]
    ns = lax.rsqrt(jnp.maximum(dsrc, 1.0))
    nd = lax.rsqrt(jnp.maximum(ddst, 1.0))
    ns_ref[0] = ns
    nd_ref[0] = nd
    xs_ref[0] = f * ns

    @pl.when((g == 0) & (i == 0))
    def _():
        r = lax.broadcasted_iota(jnp.int32, (G_ROWS, 1), 0)
        adj = jnp.where((r >= 2 * EROWS) & (r < 3 * EROWS), NPAD, 0)
        gadj_ref[...] = g2d_ref[...] + adj


def _prescale_call(feats, degs4, g2d):
    rb = 1024
    nb = NPAD // rb
    return pl.pallas_call(
        _prescale_body,
        grid=(2, nb),
        in_specs=[
            pl.BlockSpec((1, rb, D), lambda g, i: (g, i, 0)),
            pl.BlockSpec((1, 2, rb, 1), lambda g, i: (g, 0, i, 0)),
            pl.BlockSpec((G_ROWS, CH), lambda g, i: (0, 0)),
        ],
        out_specs=[
            pl.BlockSpec((1, rb, D), lambda g, i: (g, i, 0)),
            pl.BlockSpec((1, rb, 1), lambda g, i: (g, i, 0)),
            pl.BlockSpec((1, rb, 1), lambda g, i: (g, i, 0)),
            pl.BlockSpec((G_ROWS, CH), lambda g, i: (0, 0)),
        ],
        out_shape=[
            jax.ShapeDtypeStruct((2, NPAD, D), jnp.float32),
            jax.ShapeDtypeStruct((2, NPAD, 1), jnp.float32),
            jax.ShapeDtypeStruct((2, NPAD, 1), jnp.float32),
            jax.ShapeDtypeStruct((G_ROWS, CH), jnp.int32),
        ],
    )(feats, degs4, g2d)


_RB = 512
_NB = NPAD // _RB


def _mlp1_body(agg_ref, ns_ref, nd_ref, w1_ref, b1_ref, w2_ref, ys_ref):
    a = agg_ref[0]
    h = jnp.dot(a, w1_ref[...], preferred_element_type=jnp.float32)
    h = jnp.maximum(h * nd_ref[0] + b1_ref[...], 0.0)
    y = jnp.dot(h, w2_ref[...], preferred_element_type=jnp.float32)
    ys_ref[0] = y * ns_ref[0]


def _mlp1_call(agg, ns, nd, W1, b1, W2):
    return pl.pallas_call(
        _mlp1_body,
        grid=(2, _NB),
        in_specs=[
            pl.BlockSpec((1, _RB, D), lambda g, i: (g, i, 0)),
            pl.BlockSpec((1, _RB, 1), lambda g, i: (g, i, 0)),
            pl.BlockSpec((1, _RB, 1), lambda g, i: (g, i, 0)),
            pl.BlockSpec((D, H2), lambda g, i: (0, 0)),
            pl.BlockSpec((1, H2), lambda g, i: (0, 0)),
            pl.BlockSpec((H2, D), lambda g, i: (0, 0)),
        ],
        out_specs=pl.BlockSpec((1, _RB, D), lambda g, i: (g, i, 0)),
        out_shape=jax.ShapeDtypeStruct((2, NPAD, D), jnp.float32),
    )(agg, ns, nd, W1, b1, W2)


def _proj_body(agg_ref, nd_ref, b2_ref, f1w_ref, f1b_ref, f2w_ref, f2b_ref,
               zn_ref):
    i = pl.program_id(1)
    h = jnp.maximum(agg_ref[0] * nd_ref[0] + b2_ref[...], 0.0)
    t = jnp.dot(h, f1w_ref[...], preferred_element_type=jnp.float32) + f1b_ref[...]
    e = jnp.where(t > 0.0, t, jnp.exp(t) - 1.0)
    z = jnp.dot(e, f2w_ref[...], preferred_element_type=jnp.float32) + f2b_ref[...]
    nrm = jnp.sqrt(jnp.sum(z * z, axis=1, keepdims=True))
    zn = z / jnp.maximum(nrm, 1e-12)
    rows = lax.broadcasted_iota(jnp.int32, (_RB, 1), 0) + i * _RB
    zn = jnp.where(rows < N, zn, 0.0)
    zn_ref[0] = zn.astype(jnp.bfloat16)


def _proj_call(agg2, nd, b2, fc1_W, fc1_b, fc2_W, fc2_b):
    return pl.pallas_call(
        _proj_body,
        grid=(2, _NB),
        in_specs=[
            pl.BlockSpec((1, _RB, D), lambda g, i: (g, i, 0)),
            pl.BlockSpec((1, _RB, 1), lambda g, i: (g, i, 0)),
            pl.BlockSpec((1, D), lambda g, i: (0, 0)),
            pl.BlockSpec((D, D), lambda g, i: (0, 0)),
            pl.BlockSpec((1, D), lambda g, i: (0, 0)),
            pl.BlockSpec((D, D), lambda g, i: (0, 0)),
            pl.BlockSpec((1, D), lambda g, i: (0, 0)),
        ],
        out_specs=pl.BlockSpec((1, _RB, D), lambda g, i: (g, i, 0)),
        out_shape=jax.ShapeDtypeStruct((2, NPAD, D), jnp.bfloat16),
    )(agg2, nd, b2, fc1_W, fc1_b, fc2_W, fc2_b)


_RI = 512
_CJ = 2048
_NBI = NPAD // _RI
_NBJ = NPAD // _CJ
_DN = (((1,), (1,)), ((), ()))


def _loss_body(zr_ref, zc_ref, out_ref, rsA, rsB, rsC, rsD, dg):
    i = pl.program_id(0)
    j = pl.program_id(1)
    z1r = zr_ref[0]
    z2r = zr_ref[1]
    z1c = zc_ref[0]
    z2c = zc_ref[1]
    sa_m = lax.dot_general(z1r, z1c, _DN, preferred_element_type=jnp.float32)
    sb_m = lax.dot_general(z2r, z2c, _DN, preferred_element_type=jnp.float32)
    sc_m = lax.dot_general(z1r, z2c, _DN, preferred_element_type=jnp.float32)
    sd_m = lax.dot_general(z2r, z1c, _DN, preferred_element_type=jnp.float32)
    colg = lax.broadcasted_iota(jnp.int32, (_RI, _CJ), 1) + j * _CJ
    cval = colg < N
    zero = jnp.float32(0.0)
    ea = jnp.where(cval, jnp.exp(INV_T * sa_m), zero)
    eb = jnp.where(cval, jnp.exp(INV_T * sb_m), zero)
    ec = jnp.where(cval, jnp.exp(INV_T * sc_m), zero)
    ed = jnp.where(cval, jnp.exp(INV_T * sd_m), zero)
    sa = jnp.sum(ea, axis=1, keepdims=True)
    sb = jnp.sum(eb, axis=1, keepdims=True)
    sc = jnp.sum(ec, axis=1, keepdims=True)
    sd = jnp.sum(ed, axis=1, keepdims=True)
    rowg = lax.broadcasted_iota(jnp.int32, (_RI, _CJ), 0) + i * _RI
    dd = jnp.sum(jnp.where(rowg == colg, INV_T * sc_m, zero), axis=1,
                 keepdims=True)
    sl = pl.ds(i * _RI, _RI)

    @pl.when(j == 0)
    def _():
        rsA[sl, :] = sa
        rsB[sl, :] = sb
        rsC[sl, :] = sc
        rsD[sl, :] = sd
        dg[sl, :] = dd

    @pl.when(j > 0)
    def _():
        rsA[sl, :] += sa
        rsB[sl, :] += sb
        rsC[sl, :] += sc
        rsD[sl, :] += sd
        dg[sl, :] += dd

    @pl.when((i == _NBI - 1) & (j == _NBJ - 1))
    def _():
        x1 = rsA[...] + rsC[...] - E2
        x2 = rsB[...] + rsD[...] - E2
        lv = -dg[...] + 0.5 * (jnp.log(x1) + jnp.log(x2))
        rows = lax.broadcasted_iota(jnp.int32, (NPAD, 1), 0)
        tot = jnp.sum(jnp.where(rows < N, lv, zero), keepdims=True)
        out_ref[...] = tot / N


def _loss_call(zn):
    return pl.pallas_call(
        _loss_body,
        grid=(_NBI, _NBJ),
        in_specs=[
            pl.BlockSpec((2, _RI, D), lambda i, j: (0, i, 0)),
            pl.BlockSpec((2, _CJ, D), lambda i, j: (0, j, 0)),
        ],
        out_specs=pl.BlockSpec((1, 1), lambda i, j: (0, 0)),
        out_shape=jax.ShapeDtypeStruct((1, 1), jnp.float32),
        scratch_shapes=[pltpu.VMEM((NPAD, 1), jnp.float32) for _ in range(5)],
    )(zn, zn)


def kernel(graph1, graph2, feat1, feat2, W1, b1, W2, b2, fc1_W, fc1_b, fc2_W,
           fc2_b):
    g2d = jnp.concatenate([graph1.astype(jnp.int32).reshape(-1),
                           graph2.astype(jnp.int32).reshape(-1)]
                          ).reshape(G_ROWS, CH)
    f1p = jnp.pad(feat1, ((0, NPAD - N), (0, 0)))
    f2p = jnp.pad(feat2, ((0, NPAD - N), (0, 0)))
    feats = jnp.stack([f1p, f2p])
    zeros1 = jnp.zeros((NPAD,), jnp.float32)
    zeros2 = jnp.zeros((NPAD, D), jnp.float32)
    ones1 = jnp.ones((CH,), jnp.float32)

    degs = _deg_kernel(g2d.reshape(-1), zeros1, ones1)
    xs, ns, nd, g2d_adj = _prescale_call(feats, degs.reshape(2, 2, NPAD, 1),
                                         g2d)
    g_adj = g2d_adj.reshape(-1)
    agg = _segsum_kernel(xs.reshape(2 * NPAD, D), g_adj, zeros2)
    ys = _mlp1_call(agg, ns, nd, W1, b1.reshape(1, H2), W2)
    agg2 = _segsum_kernel(ys.reshape(2 * NPAD, D), g_adj, zeros2)
    zn = _proj_call(agg2, nd, b2.reshape(1, D), fc1_W, fc1_b.reshape(1, D),
                    fc2_W, fc2_b.reshape(1, D))
    out = _loss_call(zn)
    return out.reshape(())


# speedup vs baseline: 9.7357x; 1.1256x over previous
import functools

import jax
import jax.numpy as jnp
import numpy as np
from jax import lax
from jax.experimental import pallas as pl
from jax.experimental.pallas import tpu as pltpu
import jax.experimental.pallas.tpu_sc as plsc

N = 10000
NPAD = 10240
E = 320000
D = 128
H2 = 256
TEMP = 0.5
INV_T = 1.0 / TEMP
E2 = float(np.exp(1.0 / TEMP))

NT = 16
EDG_T = E // NT
CH = 40
NIT = EDG_T // CH
RPT = NPAD // NT
EROWS = E // CH
G_ROWS = 4 * EROWS

CHD = 80
NITD = EDG_T // CHD
_DRING = 5


def _deg_body(g_hbm, zeros1_hbm, ones_hbm, out_hbm, sidx, didx, ones, acc_s,
              acc_d, isem, ssem):
    c = lax.axis_index("c")
    s = lax.axis_index("s")
    pltpu.sync_copy(ones_hbm, ones)
    pltpu.sync_copy(zeros1_hbm.at[pl.ds(s * RPT, RPT)], acc_s.at[pl.ds(s * RPT, RPT)])
    pltpu.sync_copy(zeros1_hbm.at[pl.ds(s * RPT, RPT)], acc_d.at[pl.ds(s * RPT, RPT)])
    plsc.subcore_barrier()

    sbase = c * (2 * E) + s * EDG_T
    dbase = sbase + E

    def idx_start(t, b):
        pltpu.async_copy(g_hbm.at[pl.ds(sbase + t * CHD, CHD)], sidx[b], isem[b])
        pltpu.async_copy(g_hbm.at[pl.ds(dbase + t * CHD, CHD)], didx[b], isem[b])

    def idx_wait(t, b):
        pltpu.make_async_copy(g_hbm.at[pl.ds(sbase + t * CHD, CHD)], sidx[b], isem[b]).wait()
        pltpu.make_async_copy(g_hbm.at[pl.ds(dbase + t * CHD, CHD)], didx[b], isem[b]).wait()

    def scatter_start(b):
        pltpu.async_copy(ones, acc_s.at[sidx[b]], ssem[b], add=True)
        pltpu.async_copy(ones, acc_d.at[didx[b]], ssem[b], add=True)

    def scatter_wait(b):
        pltpu.make_async_copy(ones, acc_s.at[sidx[b]], ssem[b]).wait()
        pltpu.make_async_copy(ones, acc_d.at[didx[b]], ssem[b]).wait()

    idx_start(0, 0)
    idx_start(1, 1)

    def body(g, carry):
        for b in range(_DRING):
            t = _DRING * g + b
            bi = (b + 2) % _DRING

            @pl.when((t >= 3) & (t + 2 <= NITD - 1))
            def _():
                scatter_wait(bi)

            @pl.when(t + 2 <= NITD - 1)
            def _():
                idx_start(t + 2, bi)

            idx_wait(t, b)
            scatter_start(b)
        return carry

    lax.fori_loop(0, NITD // _DRING, body, 0)
    for t in range(NITD - _DRING, NITD):
        scatter_wait(t % _DRING)
    plsc.subcore_barrier()
    sl = pl.ds(s * RPT, RPT)
    pltpu.sync_copy(acc_s.at[sl], out_hbm.at[pl.ds(c * 2 * NPAD + s * RPT, RPT)])
    pltpu.sync_copy(acc_d.at[sl], out_hbm.at[pl.ds((c * 2 + 1) * NPAD + s * RPT, RPT)])


@functools.cache
def _make_deg_kernel():
    mesh = plsc.VectorSubcoreMesh(core_axis_name="c", subcore_axis_name="s",
                                  num_cores=2, num_subcores=NT)
    return pl.kernel(
        _deg_body,
        out_type=jax.ShapeDtypeStruct((4 * NPAD,), jnp.float32),
        mesh=mesh,
        scratch_types=[
            [pltpu.VMEM((CHD,), jnp.int32) for _ in range(_DRING)],
            [pltpu.VMEM((CHD,), jnp.int32) for _ in range(_DRING)],
            pltpu.VMEM((CHD,), jnp.float32),
            pltpu.VMEM_SHARED((NPAD,), jnp.float32),
            pltpu.VMEM_SHARED((NPAD,), jnp.float32),
            [pltpu.SemaphoreType.DMA for _ in range(_DRING)],
            [pltpu.SemaphoreType.DMA for _ in range(_DRING)],
        ],
    )


def _deg_kernel(g_flat, zeros1, ones1):
    return _make_deg_kernel()(g_flat, zeros1, ones1)


_RING = 5


def _segsum_body(xs_hbm, g_hbm, zeros2_hbm, out_hbm, sidx, didx, rows, acc,
                 isem, gsem, ssem):
    c = lax.axis_index("c")
    s = lax.axis_index("s")
    pltpu.sync_copy(zeros2_hbm.at[pl.ds(s * RPT, RPT)], acc.at[pl.ds(s * RPT, RPT)])
    plsc.subcore_barrier()

    sbase = c * (2 * E) + s * EDG_T
    dbase = sbase + E

    def idx_start(t, b):
        pltpu.async_copy(g_hbm.at[pl.ds(sbase + t * CH, CH)], sidx[b], isem[b])
        pltpu.async_copy(g_hbm.at[pl.ds(dbase + t * CH, CH)], didx[b], isem[b])

    def idx_wait(t, b):
        pltpu.make_async_copy(g_hbm.at[pl.ds(sbase + t * CH, CH)], sidx[b], isem[b]).wait()
        pltpu.make_async_copy(g_hbm.at[pl.ds(dbase + t * CH, CH)], didx[b], isem[b]).wait()

    def gather_start(b):
        pltpu.async_copy(xs_hbm.at[sidx[b]], rows[b], gsem[b])

    def gather_wait(b):
        pltpu.make_async_copy(xs_hbm.at[sidx[b]], rows[b], gsem[b]).wait()

    def scatter_start(b):
        pltpu.async_copy(rows[b], acc.at[didx[b]], ssem[b], add=True)

    def scatter_wait(b):
        pltpu.make_async_copy(rows[b], acc.at[didx[b]], ssem[b]).wait()

    idx_start(0, 0)
    idx_start(1, 1)
    idx_wait(0, 0)
    gather_start(0)

    def body(g, carry):
        for b in range(_RING):
            t = _RING * g + b
            bi = (b + 2) % _RING

            @pl.when((t >= 3) & (t + 2 <= NIT - 1))
            def _():
                scatter_wait(bi)

            @pl.when(t + 2 <= NIT - 1)
            def _():
                idx_start(t + 2, bi)

            bg = (b + 1) % _RING

            @pl.when(t + 1 <= NIT - 1)
            def _():
                idx_wait(t + 1, bg)
                gather_start(bg)

            gather_wait(b)
            scatter_start(b)
        return carry

    lax.fori_loop(0, NIT // _RING, body, 0)
    for t in range(NIT - _RING, NIT):
        scatter_wait(t % _RING)
    plsc.subcore_barrier()
    sl = pl.ds(s * RPT, RPT)
    pltpu.sync_copy(acc.at[sl], out_hbm.at[c, sl])


@functools.cache
def _make_segsum_kernel():
    mesh = plsc.VectorSubcoreMesh(core_axis_name="c", subcore_axis_name="s",
                                  num_cores=2, num_subcores=NT)
    return pl.kernel(
        _segsum_body,
        out_type=jax.ShapeDtypeStruct((2, NPAD, D), jnp.float32),
        mesh=mesh,
        scratch_types=[
            [pltpu.VMEM((CH,), jnp.int32) for _ in range(_RING)],
            [pltpu.VMEM((CH,), jnp.int32) for _ in range(_RING)],
            [pltpu.VMEM((CH, D), jnp.float32) for _ in range(_RING)],
            pltpu.VMEM_SHARED((NPAD, D), jnp.float32),
            [pltpu.SemaphoreType.DMA for _ in range(_RING)],
            [pltpu.SemaphoreType.DMA for _ in range(_RING)],
            [pltpu.SemaphoreType.DMA for _ in range(_RING)],
        ],
    )


def _segsum_kernel(xs_flat, g_adj_flat, zeros2):
    return _make_segsum_kernel()(xs_flat, g_adj_flat, zeros2)


def _prescale_body(feat_ref, deg_ref, g2d_ref, xs_ref, ns_ref, nd_ref,
                   gadj_ref):
    g = pl.program_id(0)
    i = pl.program_id(1)
    f = feat_ref[0]
    dsrc = deg_ref[0, 0]
    ddst = deg_ref[0, 1]
    ns = lax.rsqrt(jnp.maximum(dsrc, 1.0))
    nd = lax.rsqrt(jnp.maximum(ddst, 1.0))
    ns_ref[0] = ns
    nd_ref[0] = nd
    xs_ref[0] = f * ns

    @pl.when((g == 0) & (i == 0))
    def _():
        r = lax.broadcasted_iota(jnp.int32, (G_ROWS, 1), 0)
        adj = jnp.where((r >= 2 * EROWS) & (r < 3 * EROWS), NPAD, 0)
        gadj_ref[...] = g2d_ref[...] + adj


def _prescale_call(feats, degs4, g2d):
    rb = 1024
    nb = NPAD // rb
    return pl.pallas_call(
        _prescale_body,
        grid=(2, nb),
        in_specs=[
            pl.BlockSpec((1, rb, D), lambda g, i: (g, i, 0)),
            pl.BlockSpec((1, 2, rb, 1), lambda g, i: (g, 0, i, 0)),
            pl.BlockSpec((G_ROWS, CH), lambda g, i: (0, 0)),
        ],
        out_specs=[
            pl.BlockSpec((1, rb, D), lambda g, i: (g, i, 0)),
            pl.BlockSpec((1, rb, 1), lambda g, i: (g, i, 0)),
            pl.BlockSpec((1, rb, 1), lambda g, i: (g, i, 0)),
            pl.BlockSpec((G_ROWS, CH), lambda g, i: (0, 0)),
        ],
        out_shape=[
            jax.ShapeDtypeStruct((2, NPAD, D), jnp.float32),
            jax.ShapeDtypeStruct((2, NPAD, 1), jnp.float32),
            jax.ShapeDtypeStruct((2, NPAD, 1), jnp.float32),
            jax.ShapeDtypeStruct((G_ROWS, CH), jnp.int32),
        ],
    )(feats, degs4, g2d)


_RB = 512
_NB = NPAD // _RB


def _mlp1_body(agg_ref, ns_ref, nd_ref, w1_ref, b1_ref, w2_ref, ys_ref):
    a = agg_ref[0]
    h = jnp.dot(a, w1_ref[...], preferred_element_type=jnp.float32)
    h = jnp.maximum(h * nd_ref[0] + b1_ref[...], 0.0)
    y = jnp.dot(h, w2_ref[...], preferred_element_type=jnp.float32)
    ys_ref[0] = y * ns_ref[0]


def _mlp1_call(agg, ns, nd, W1, b1, W2):
    return pl.pallas_call(
        _mlp1_body,
        grid=(2, _NB),
        in_specs=[
            pl.BlockSpec((1, _RB, D), lambda g, i: (g, i, 0)),
            pl.BlockSpec((1, _RB, 1), lambda g, i: (g, i, 0)),
            pl.BlockSpec((1, _RB, 1), lambda g, i: (g, i, 0)),
            pl.BlockSpec((D, H2), lambda g, i: (0, 0)),
            pl.BlockSpec((1, H2), lambda g, i: (0, 0)),
            pl.BlockSpec((H2, D), lambda g, i: (0, 0)),
        ],
        out_specs=pl.BlockSpec((1, _RB, D), lambda g, i: (g, i, 0)),
        out_shape=jax.ShapeDtypeStruct((2, NPAD, D), jnp.float32),
    )(agg, ns, nd, W1, b1, W2)


def _proj_body(agg_ref, nd_ref, b2_ref, f1w_ref, f1b_ref, f2w_ref, f2b_ref,
               zn_ref):
    i = pl.program_id(1)
    h = jnp.maximum(agg_ref[0] * nd_ref[0] + b2_ref[...], 0.0)
    t = jnp.dot(h, f1w_ref[...], preferred_element_type=jnp.float32) + f1b_ref[...]
    e = jnp.where(t > 0.0, t, jnp.exp(t) - 1.0)
    z = jnp.dot(e, f2w_ref[...], preferred_element_type=jnp.float32) + f2b_ref[...]
    nrm = jnp.sqrt(jnp.sum(z * z, axis=1, keepdims=True))
    zn = z / jnp.maximum(nrm, 1e-12)
    rows = lax.broadcasted_iota(jnp.int32, (_RB, 1), 0) + i * _RB
    zn = jnp.where(rows < N, zn, 0.0)
    zn_ref[0] = zn.astype(jnp.bfloat16)


def _proj_call(agg2, nd, b2, fc1_W, fc1_b, fc2_W, fc2_b):
    return pl.pallas_call(
        _proj_body,
        grid=(2, _NB),
        in_specs=[
            pl.BlockSpec((1, _RB, D), lambda g, i: (g, i, 0)),
            pl.BlockSpec((1, _RB, 1), lambda g, i: (g, i, 0)),
            pl.BlockSpec((1, D), lambda g, i: (0, 0)),
            pl.BlockSpec((D, D), lambda g, i: (0, 0)),
            pl.BlockSpec((1, D), lambda g, i: (0, 0)),
            pl.BlockSpec((D, D), lambda g, i: (0, 0)),
            pl.BlockSpec((1, D), lambda g, i: (0, 0)),
        ],
        out_specs=pl.BlockSpec((1, _RB, D), lambda g, i: (g, i, 0)),
        out_shape=jax.ShapeDtypeStruct((2, NPAD, D), jnp.bfloat16),
    )(agg2, nd, b2, fc1_W, fc1_b, fc2_W, fc2_b)


_RI = 512
_CJ = 2048
_NBI = NPAD // _RI
_NBJ = NPAD // _CJ
_DN = (((1,), (1,)), ((), ()))


def _loss_body(zr_ref, zc_ref, out_ref, rsA, rsB, rsC, csC, dg):
    i = pl.program_id(0)
    j = pl.program_id(1)
    z1r = zr_ref[0]
    z2r = zr_ref[1]
    z1c = zc_ref[0]
    z2c = zc_ref[1]
    sa_m = lax.dot_general(z1r, z1c, _DN, preferred_element_type=jnp.float32)
    sb_m = lax.dot_general(z2r, z2c, _DN, preferred_element_type=jnp.float32)
    sc_m = lax.dot_general(z1r, z2c, _DN, preferred_element_type=jnp.float32)
    colg = lax.broadcasted_iota(jnp.int32, (_RI, _CJ), 1) + j * _CJ
    rowg = lax.broadcasted_iota(jnp.int32, (_RI, _CJ), 0) + i * _RI
    cval = colg < N
    zero = jnp.float32(0.0)
    ea = jnp.where(cval, jnp.exp(INV_T * sa_m), zero)
    eb = jnp.where(cval, jnp.exp(INV_T * sb_m), zero)
    ec = jnp.where(cval & (rowg < N), jnp.exp(INV_T * sc_m), zero)
    sa = jnp.sum(ea, axis=1, keepdims=True)
    sb = jnp.sum(eb, axis=1, keepdims=True)
    sc = jnp.sum(ec, axis=1, keepdims=True)
    cs = jnp.sum(ec, axis=0, keepdims=True)
    dd = jnp.sum(jnp.where(rowg == colg, INV_T * sc_m, zero), axis=1,
                 keepdims=True)
    sl = pl.ds(i * _RI, _RI)

    @pl.when(j == 0)
    def _():
        rsA[sl, :] = sa
        rsB[sl, :] = sb
        rsC[sl, :] = sc
        dg[sl, :] = dd

    @pl.when(j > 0)
    def _():
        rsA[sl, :] += sa
        rsB[sl, :] += sb
        rsC[sl, :] += sc
        dg[sl, :] += dd

    @pl.when(i == 0)
    def _():
        csC[j, :, :] = cs

    @pl.when(i > 0)
    def _():
        csC[j, :, :] += cs

    @pl.when((i == _NBI - 1) & (j == _NBJ - 1))
    def _():
        x1 = rsA[...] + rsC[...] - E2
        lv1 = -dg[...] + 0.5 * jnp.log(x1)
        rows = lax.broadcasted_iota(jnp.int32, (NPAD, 1), 0)
        tot = jnp.sum(jnp.where(rows < N, lv1, zero), keepdims=True)
        rows_j = lax.broadcasted_iota(jnp.int32, (_CJ, 1), 0)
        for jj in range(_NBJ):
            col = jnp.transpose(csC[jj, :, :])
            x2 = rsB[pl.ds(jj * _CJ, _CJ), :] + col - E2
            lv2 = 0.5 * jnp.log(x2)
            val = (rows_j + jj * _CJ) < N
            tot = tot + jnp.sum(jnp.where(val, lv2, zero), keepdims=True)
        out_ref[...] = tot / N


def _loss_call(zn):
    return pl.pallas_call(
        _loss_body,
        grid=(_NBI, _NBJ),
        in_specs=[
            pl.BlockSpec((2, _RI, D), lambda i, j: (0, i, 0)),
            pl.BlockSpec((2, _CJ, D), lambda i, j: (0, j, 0)),
        ],
        out_specs=pl.BlockSpec((1, 1), lambda i, j: (0, 0)),
        out_shape=jax.ShapeDtypeStruct((1, 1), jnp.float32),
        scratch_shapes=[
            pltpu.VMEM((NPAD, 1), jnp.float32),
            pltpu.VMEM((NPAD, 1), jnp.float32),
            pltpu.VMEM((NPAD, 1), jnp.float32),
            pltpu.VMEM((_NBJ, 1, _CJ), jnp.float32),
            pltpu.VMEM((NPAD, 1), jnp.float32),
        ],
    )(zn, zn)


def kernel(graph1, graph2, feat1, feat2, W1, b1, W2, b2, fc1_W, fc1_b, fc2_W,
           fc2_b):
    g2d = jnp.concatenate([graph1.astype(jnp.int32).reshape(-1),
                           graph2.astype(jnp.int32).reshape(-1)]
                          ).reshape(G_ROWS, CH)
    f1p = jnp.pad(feat1, ((0, NPAD - N), (0, 0)))
    f2p = jnp.pad(feat2, ((0, NPAD - N), (0, 0)))
    feats = jnp.stack([f1p, f2p])
    zeros1 = jnp.zeros((NPAD,), jnp.float32)
    zeros2 = jnp.zeros((NPAD, D), jnp.float32)
    ones1 = jnp.ones((CHD,), jnp.float32)

    degs = _deg_kernel(g2d.reshape(-1), zeros1, ones1)
    xs, ns, nd, g2d_adj = _prescale_call(feats, degs.reshape(2, 2, NPAD, 1),
                                         g2d)
    g_adj = g2d_adj.reshape(-1)
    agg = _segsum_kernel(xs.reshape(2 * NPAD, D), g_adj, zeros2)
    ys = _mlp1_call(agg, ns, nd, W1, b1.reshape(1, H2), W2)
    agg2 = _segsum_kernel(ys.reshape(2 * NPAD, D), g_adj, zeros2)
    zn = _proj_call(agg2, nd, b2.reshape(1, D), fc1_W, fc1_b.reshape(1, D),
                    fc2_W, fc2_b.reshape(1, D))
    out = _loss_call(zn)
    return out.reshape(())


# speedup vs baseline: 9.9056x; 1.0175x over previous
import functools

import jax
import jax.numpy as jnp
import numpy as np
from jax import lax
from jax.experimental import pallas as pl
from jax.experimental.pallas import tpu as pltpu
import jax.experimental.pallas.tpu_sc as plsc

N = 10000
NPAD = 10240
E = 320000
D = 128
H2 = 256
TEMP = 0.5
INV_T = 1.0 / TEMP
E2 = float(np.exp(1.0 / TEMP))

NT = 16
EDG_T = E // NT
CH = 40
NIT = EDG_T // CH
RPT = NPAD // NT
EROWS = E // CH
G_ROWS = 4 * EROWS

CHD = 80
NITD = EDG_T // CHD
_DRING = 5


def _deg_body(g_hbm, zeros1_hbm, ones_hbm, out_hbm, sidx, didx, ones, acc_s,
              acc_d, isem, ssem):
    c = lax.axis_index("c")
    s = lax.axis_index("s")
    pltpu.sync_copy(ones_hbm, ones)
    pltpu.sync_copy(zeros1_hbm.at[pl.ds(s * RPT, RPT)], acc_s.at[pl.ds(s * RPT, RPT)])
    pltpu.sync_copy(zeros1_hbm.at[pl.ds(s * RPT, RPT)], acc_d.at[pl.ds(s * RPT, RPT)])
    plsc.subcore_barrier()

    sbase = c * (2 * E) + s * EDG_T
    dbase = sbase + E

    def idx_start(t, b):
        pltpu.async_copy(g_hbm.at[pl.ds(sbase + t * CHD, CHD)], sidx[b], isem[b])
        pltpu.async_copy(g_hbm.at[pl.ds(dbase + t * CHD, CHD)], didx[b], isem[b])

    def idx_wait(t, b):
        pltpu.make_async_copy(g_hbm.at[pl.ds(sbase + t * CHD, CHD)], sidx[b], isem[b]).wait()
        pltpu.make_async_copy(g_hbm.at[pl.ds(dbase + t * CHD, CHD)], didx[b], isem[b]).wait()

    def scatter_start(b):
        pltpu.async_copy(ones, acc_s.at[sidx[b]], ssem[b], add=True)
        pltpu.async_copy(ones, acc_d.at[didx[b]], ssem[b], add=True)

    def scatter_wait(b):
        pltpu.make_async_copy(ones, acc_s.at[sidx[b]], ssem[b]).wait()
        pltpu.make_async_copy(ones, acc_d.at[didx[b]], ssem[b]).wait()

    idx_start(0, 0)
    idx_start(1, 1)

    def body(g, carry):
        for b in range(_DRING):
            t = _DRING * g + b
            bi = (b + 2) % _DRING

            @pl.when((t >= 3) & (t + 2 <= NITD - 1))
            def _():
                scatter_wait(bi)

            @pl.when(t + 2 <= NITD - 1)
            def _():
                idx_start(t + 2, bi)

            idx_wait(t, b)
            scatter_start(b)
        return carry

    lax.fori_loop(0, NITD // _DRING, body, 0)
    for t in range(NITD - _DRING, NITD):
        scatter_wait(t % _DRING)
    plsc.subcore_barrier()
    sl = pl.ds(s * RPT, RPT)
    pltpu.sync_copy(acc_s.at[sl], out_hbm.at[pl.ds(c * 2 * NPAD + s * RPT, RPT)])
    pltpu.sync_copy(acc_d.at[sl], out_hbm.at[pl.ds((c * 2 + 1) * NPAD + s * RPT, RPT)])


@functools.cache
def _make_deg_kernel():
    mesh = plsc.VectorSubcoreMesh(core_axis_name="c", subcore_axis_name="s",
                                  num_cores=2, num_subcores=NT)
    return pl.kernel(
        _deg_body,
        out_type=jax.ShapeDtypeStruct((4 * NPAD,), jnp.float32),
        mesh=mesh,
        scratch_types=[
            [pltpu.VMEM((CHD,), jnp.int32) for _ in range(_DRING)],
            [pltpu.VMEM((CHD,), jnp.int32) for _ in range(_DRING)],
            pltpu.VMEM((CHD,), jnp.float32),
            pltpu.VMEM_SHARED((NPAD,), jnp.float32),
            pltpu.VMEM_SHARED((NPAD,), jnp.float32),
            [pltpu.SemaphoreType.DMA for _ in range(_DRING)],
            [pltpu.SemaphoreType.DMA for _ in range(_DRING)],
        ],
    )


def _deg_kernel(g_flat, zeros1, ones1):
    return _make_deg_kernel()(g_flat, zeros1, ones1)


_RING = 5
EDG_T1 = E // (2 * NT)
NIT1 = EDG_T1 // CH


def _make_segsum_body(goff):
    def _segsum_body(xs_hbm, g_hbm, zeros2_hbm, out_hbm, sidx, didx, rows,
                     acc, isem, gsem, ssem):
        c = lax.axis_index("c")
        s = lax.axis_index("s")
        pltpu.sync_copy(zeros2_hbm.at[pl.ds(s * RPT, RPT)], acc.at[pl.ds(s * RPT, RPT)])
        plsc.subcore_barrier()

        w = c * NT + s
        sbase = goff * (2 * E) + w * EDG_T1
        dbase = sbase + E

        def idx_start(t, b):
            pltpu.async_copy(g_hbm.at[pl.ds(sbase + t * CH, CH)], sidx[b], isem[b])
            pltpu.async_copy(g_hbm.at[pl.ds(dbase + t * CH, CH)], didx[b], isem[b])

        def idx_wait(t, b):
            pltpu.make_async_copy(g_hbm.at[pl.ds(sbase + t * CH, CH)], sidx[b], isem[b]).wait()
            pltpu.make_async_copy(g_hbm.at[pl.ds(dbase + t * CH, CH)], didx[b], isem[b]).wait()

        def gather_start(b):
            pltpu.async_copy(xs_hbm.at[sidx[b]], rows[b], gsem[b])

        def gather_wait(b):
            pltpu.make_async_copy(xs_hbm.at[sidx[b]], rows[b], gsem[b]).wait()

        def scatter_start(b):
            pltpu.async_copy(rows[b], acc.at[didx[b]], ssem[b], add=True)

        def scatter_wait(b):
            pltpu.make_async_copy(rows[b], acc.at[didx[b]], ssem[b]).wait()

        idx_start(0, 0)
        idx_start(1, 1)
        idx_wait(0, 0)
        gather_start(0)

        def body(g, carry):
            for b in range(_RING):
                t = _RING * g + b
                bi = (b + 2) % _RING

                @pl.when((t >= 3) & (t + 2 <= NIT1 - 1))
                def _():
                    scatter_wait(bi)

                @pl.when(t + 2 <= NIT1 - 1)
                def _():
                    idx_start(t + 2, bi)

                bg = (b + 1) % _RING

                @pl.when(t + 1 <= NIT1 - 1)
                def _():
                    idx_wait(t + 1, bg)
                    gather_start(bg)

                gather_wait(b)
                scatter_start(b)
            return carry

        lax.fori_loop(0, NIT1 // _RING, body, 0)
        for t in range(NIT1 - _RING, NIT1):
            scatter_wait(t % _RING)
        plsc.subcore_barrier()
        sl = pl.ds(s * RPT, RPT)
        pltpu.sync_copy(acc.at[sl], out_hbm.at[c, sl])

    return _segsum_body


@functools.cache
def _make_segsum_kernel(goff):
    mesh = plsc.VectorSubcoreMesh(core_axis_name="c", subcore_axis_name="s",
                                  num_cores=2, num_subcores=NT)
    return pl.kernel(
        _make_segsum_body(goff),
        out_type=jax.ShapeDtypeStruct((2, NPAD, D), jnp.float32),
        mesh=mesh,
        scratch_types=[
            [pltpu.VMEM((CH,), jnp.int32) for _ in range(_RING)],
            [pltpu.VMEM((CH,), jnp.int32) for _ in range(_RING)],
            [pltpu.VMEM((CH, D), jnp.float32) for _ in range(_RING)],
            pltpu.VMEM_SHARED((NPAD, D), jnp.float32),
            [pltpu.SemaphoreType.DMA for _ in range(_RING)],
            [pltpu.SemaphoreType.DMA for _ in range(_RING)],
            [pltpu.SemaphoreType.DMA for _ in range(_RING)],
        ],
    )


def _segsum_kernel(xs_g, g_flat, zeros2, goff):
    return _make_segsum_kernel(goff)(xs_g, g_flat, zeros2)


def _prescale_body(feat_ref, deg_ref, xs_ref, ns_ref, nd_ref):
    f = feat_ref[0]
    dsrc = deg_ref[0, 0]
    ddst = deg_ref[0, 1]
    ns = lax.rsqrt(jnp.maximum(dsrc, 1.0))
    nd = lax.rsqrt(jnp.maximum(ddst, 1.0))
    ns_ref[0] = ns
    nd_ref[0] = nd
    xs_ref[0] = f * ns


def _prescale_call(feats, degs4):
    rb = 1024
    nb = NPAD // rb
    return pl.pallas_call(
        _prescale_body,
        grid=(2, nb),
        in_specs=[
            pl.BlockSpec((1, rb, D), lambda g, i: (g, i, 0)),
            pl.BlockSpec((1, 2, rb, 1), lambda g, i: (g, 0, i, 0)),
        ],
        out_specs=[
            pl.BlockSpec((1, rb, D), lambda g, i: (g, i, 0)),
            pl.BlockSpec((1, rb, 1), lambda g, i: (g, i, 0)),
            pl.BlockSpec((1, rb, 1), lambda g, i: (g, i, 0)),
        ],
        out_shape=[
            jax.ShapeDtypeStruct((2, NPAD, D), jnp.float32),
            jax.ShapeDtypeStruct((2, NPAD, 1), jnp.float32),
            jax.ShapeDtypeStruct((2, NPAD, 1), jnp.float32),
        ],
    )(feats, degs4)


_RB = 512
_NB = NPAD // _RB


def _mlp1_body(agg_ref, ns_ref, nd_ref, w1_ref, b1_ref, w2_ref, ys_ref):
    a = agg_ref[0] + agg_ref[1]
    h = jnp.dot(a, w1_ref[...], preferred_element_type=jnp.float32)
    h = jnp.maximum(h * nd_ref[...] + b1_ref[...], 0.0)
    y = jnp.dot(h, w2_ref[...], preferred_element_type=jnp.float32)
    ys_ref[...] = y * ns_ref[...]


def _mlp1_call(agg, ns_g, nd_g, W1, b1, W2):
    return pl.pallas_call(
        _mlp1_body,
        grid=(_NB,),
        in_specs=[
            pl.BlockSpec((2, _RB, D), lambda i: (0, i, 0)),
            pl.BlockSpec((_RB, 1), lambda i: (i, 0)),
            pl.BlockSpec((_RB, 1), lambda i: (i, 0)),
            pl.BlockSpec((D, H2), lambda i: (0, 0)),
            pl.BlockSpec((1, H2), lambda i: (0, 0)),
            pl.BlockSpec((H2, D), lambda i: (0, 0)),
        ],
        out_specs=pl.BlockSpec((_RB, D), lambda i: (i, 0)),
        out_shape=jax.ShapeDtypeStruct((NPAD, D), jnp.float32),
    )(agg, ns_g, nd_g, W1, b1, W2)


def _proj_body(agg_ref, nd_ref, b2_ref, f1w_ref, f1b_ref, f2w_ref, f2b_ref,
               zn_ref):
    i = pl.program_id(0)
    a = agg_ref[0] + agg_ref[1]
    h = jnp.maximum(a * nd_ref[...] + b2_ref[...], 0.0)
    t = jnp.dot(h, f1w_ref[...], preferred_element_type=jnp.float32) + f1b_ref[...]
    e = jnp.where(t > 0.0, t, jnp.exp(t) - 1.0)
    z = jnp.dot(e, f2w_ref[...], preferred_element_type=jnp.float32) + f2b_ref[...]
    nrm = jnp.sqrt(jnp.sum(z * z, axis=1, keepdims=True))
    zn = z / jnp.maximum(nrm, 1e-12)
    rows = lax.broadcasted_iota(jnp.int32, (_RB, 1), 0) + i * _RB
    zn = jnp.where(rows < N, zn, 0.0)
    zn_ref[...] = zn.astype(jnp.bfloat16)


def _proj_call(agg2, nd_g, b2, fc1_W, fc1_b, fc2_W, fc2_b):
    return pl.pallas_call(
        _proj_body,
        grid=(_NB,),
        in_specs=[
            pl.BlockSpec((2, _RB, D), lambda i: (0, i, 0)),
            pl.BlockSpec((_RB, 1), lambda i: (i, 0)),
            pl.BlockSpec((1, D), lambda i: (0, 0)),
            pl.BlockSpec((D, D), lambda i: (0, 0)),
            pl.BlockSpec((1, D), lambda i: (0, 0)),
            pl.BlockSpec((D, D), lambda i: (0, 0)),
            pl.BlockSpec((1, D), lambda i: (0, 0)),
        ],
        out_specs=pl.BlockSpec((_RB, D), lambda i: (i, 0)),
        out_shape=jax.ShapeDtypeStruct((NPAD, D), jnp.bfloat16),
    )(agg2, nd_g, b2, fc1_W, fc1_b, fc2_W, fc2_b)


_RI = 512
_CJ = 2048
_NBI = NPAD // _RI
_NBJ = NPAD // _CJ
_DN = (((1,), (1,)), ((), ()))


def _loss_body(zr_ref, zc_ref, out_ref, rsA, rsB, rsC, csC, dg):
    i = pl.program_id(0)
    j = pl.program_id(1)
    z1r = zr_ref[0]
    z2r = zr_ref[1]
    z1c = zc_ref[0]
    z2c = zc_ref[1]
    sa_m = lax.dot_general(z1r, z1c, _DN, preferred_element_type=jnp.float32)
    sb_m = lax.dot_general(z2r, z2c, _DN, preferred_element_type=jnp.float32)
    sc_m = lax.dot_general(z1r, z2c, _DN, preferred_element_type=jnp.float32)
    colg = lax.broadcasted_iota(jnp.int32, (_RI, _CJ), 1) + j * _CJ
    rowg = lax.broadcasted_iota(jnp.int32, (_RI, _CJ), 0) + i * _RI
    cval = colg < N
    zero = jnp.float32(0.0)
    ea = jnp.where(cval, jnp.exp(INV_T * sa_m), zero)
    eb = jnp.where(cval, jnp.exp(INV_T * sb_m), zero)
    ec = jnp.where(cval & (rowg < N), jnp.exp(INV_T * sc_m), zero)
    sa = jnp.sum(ea, axis=1, keepdims=True)
    sb = jnp.sum(eb, axis=1, keepdims=True)
    sc = jnp.sum(ec, axis=1, keepdims=True)
    cs = jnp.sum(ec, axis=0, keepdims=True)
    dd = jnp.sum(jnp.where(rowg == colg, INV_T * sc_m, zero), axis=1,
                 keepdims=True)
    sl = pl.ds(i * _RI, _RI)

    @pl.when(j == 0)
    def _():
        rsA[sl, :] = sa
        rsB[sl, :] = sb
        rsC[sl, :] = sc
        dg[sl, :] = dd

    @pl.when(j > 0)
    def _():
        rsA[sl, :] += sa
        rsB[sl, :] += sb
        rsC[sl, :] += sc
        dg[sl, :] += dd

    @pl.when(i == 0)
    def _():
        csC[j, :, :] = cs

    @pl.when(i > 0)
    def _():
        csC[j, :, :] += cs

    @pl.when((i == _NBI - 1) & (j == _NBJ - 1))
    def _():
        x1 = rsA[...] + rsC[...] - E2
        lv1 = -dg[...] + 0.5 * jnp.log(x1)
        rows = lax.broadcasted_iota(jnp.int32, (NPAD, 1), 0)
        tot = jnp.sum(jnp.where(rows < N, lv1, zero), keepdims=True)
        rows_j = lax.broadcasted_iota(jnp.int32, (_CJ, 1), 0)
        for jj in range(_NBJ):
            col = jnp.transpose(csC[jj, :, :])
            x2 = rsB[pl.ds(jj * _CJ, _CJ), :] + col - E2
            lv2 = 0.5 * jnp.log(x2)
            val = (rows_j + jj * _CJ) < N
            tot = tot + jnp.sum(jnp.where(val, lv2, zero), keepdims=True)
        out_ref[...] = tot / N


def _loss_call(zn):
    return pl.pallas_call(
        _loss_body,
        grid=(_NBI, _NBJ),
        in_specs=[
            pl.BlockSpec((2, _RI, D), lambda i, j: (0, i, 0)),
            pl.BlockSpec((2, _CJ, D), lambda i, j: (0, j, 0)),
        ],
        out_specs=pl.BlockSpec((1, 1), lambda i, j: (0, 0)),
        out_shape=jax.ShapeDtypeStruct((1, 1), jnp.float32),
        scratch_shapes=[
            pltpu.VMEM((NPAD, 1), jnp.float32),
            pltpu.VMEM((NPAD, 1), jnp.float32),
            pltpu.VMEM((NPAD, 1), jnp.float32),
            pltpu.VMEM((_NBJ, 1, _CJ), jnp.float32),
            pltpu.VMEM((NPAD, 1), jnp.float32),
        ],
    )(zn, zn)


def kernel(graph1, graph2, feat1, feat2, W1, b1, W2, b2, fc1_W, fc1_b, fc2_W,
           fc2_b):
    g_flat = jnp.concatenate([graph1.astype(jnp.int32).reshape(-1),
                              graph2.astype(jnp.int32).reshape(-1)])
    f1p = jnp.pad(feat1, ((0, NPAD - N), (0, 0)))
    f2p = jnp.pad(feat2, ((0, NPAD - N), (0, 0)))
    feats = jnp.stack([f1p, f2p])
    zeros1 = jnp.zeros((NPAD,), jnp.float32)
    zeros2 = jnp.zeros((NPAD, D), jnp.float32)
    ones1 = jnp.ones((CHD,), jnp.float32)
    b1r = b1.reshape(1, H2)
    b2r = b2.reshape(1, D)
    f1br = fc1_b.reshape(1, D)
    f2br = fc2_b.reshape(1, D)

    degs = _deg_kernel(g_flat, zeros1, ones1)
    xs, ns, nd = _prescale_call(feats, degs.reshape(2, 2, NPAD, 1))
    a11 = _segsum_kernel(xs[0], g_flat, zeros2, 0)
    a12 = _segsum_kernel(xs[1], g_flat, zeros2, 1)
    ys1 = _mlp1_call(a11, ns[0], nd[0], W1, b1r, W2)
    ys2 = _mlp1_call(a12, ns[1], nd[1], W1, b1r, W2)
    a21 = _segsum_kernel(ys1, g_flat, zeros2, 0)
    a22 = _segsum_kernel(ys2, g_flat, zeros2, 1)
    zn1 = _proj_call(a21, nd[0], b2r, fc1_W, f1br, fc2_W, f2br)
    zn2 = _proj_call(a22, nd[1], b2r, fc1_W, f1br, fc2_W, f2br)
    zn = jnp.stack([zn1, zn2])
    out = _loss_call(zn)
    return out.reshape(())


# speedup vs baseline: 10.6887x; 1.0790x over previous
import functools

import jax
import jax.numpy as jnp
import numpy as np
from jax import lax
from jax.experimental import pallas as pl
from jax.experimental.pallas import tpu as pltpu
import jax.experimental.pallas.tpu_sc as plsc

N = 10000
NPAD = 10240
E = 320000
D = 128
H2 = 256
TEMP = 0.5
INV_T = 1.0 / TEMP
E2 = float(np.exp(1.0 / TEMP))

NT = 16
EDG_T = E // NT
CH = 40
NIT = EDG_T // CH
RPT = NPAD // NT
EROWS = E // CH
G_ROWS = 4 * EROWS

CHD = 80
NITD = EDG_T // CHD
_DRING = 5


def _deg_body(g_hbm, zeros1_hbm, ones_hbm, out_hbm, sidx, didx, ones, acc_s,
              acc_d, isem, ssem):
    c = lax.axis_index("c")
    s = lax.axis_index("s")
    pltpu.sync_copy(ones_hbm, ones)
    pltpu.sync_copy(zeros1_hbm.at[pl.ds(s * RPT, RPT)], acc_s.at[pl.ds(s * RPT, RPT)])
    pltpu.sync_copy(zeros1_hbm.at[pl.ds(s * RPT, RPT)], acc_d.at[pl.ds(s * RPT, RPT)])
    plsc.subcore_barrier()

    sbase = c * (2 * E) + s * EDG_T
    dbase = sbase + E

    def idx_start(t, b):
        pltpu.async_copy(g_hbm.at[pl.ds(sbase + t * CHD, CHD)], sidx[b], isem[b])
        pltpu.async_copy(g_hbm.at[pl.ds(dbase + t * CHD, CHD)], didx[b], isem[b])

    def idx_wait(t, b):
        pltpu.make_async_copy(g_hbm.at[pl.ds(sbase + t * CHD, CHD)], sidx[b], isem[b]).wait()
        pltpu.make_async_copy(g_hbm.at[pl.ds(dbase + t * CHD, CHD)], didx[b], isem[b]).wait()

    def scatter_start(b):
        pltpu.async_copy(ones, acc_s.at[sidx[b]], ssem[b], add=True)
        pltpu.async_copy(ones, acc_d.at[didx[b]], ssem[b], add=True)

    def scatter_wait(b):
        pltpu.make_async_copy(ones, acc_s.at[sidx[b]], ssem[b]).wait()
        pltpu.make_async_copy(ones, acc_d.at[didx[b]], ssem[b]).wait()

    idx_start(0, 0)
    idx_start(1, 1)

    def body(g, carry):
        for b in range(_DRING):
            t = _DRING * g + b
            bi = (b + 2) % _DRING

            @pl.when((t >= 3) & (t + 2 <= NITD - 1))
            def _():
                scatter_wait(bi)

            @pl.when(t + 2 <= NITD - 1)
            def _():
                idx_start(t + 2, bi)

            idx_wait(t, b)
            scatter_start(b)
        return carry

    lax.fori_loop(0, NITD // _DRING, body, 0)
    for t in range(NITD - _DRING, NITD):
        scatter_wait(t % _DRING)
    plsc.subcore_barrier()
    sl = pl.ds(s * RPT, RPT)
    pltpu.sync_copy(acc_s.at[sl], out_hbm.at[pl.ds(c * 2 * NPAD + s * RPT, RPT)])
    pltpu.sync_copy(acc_d.at[sl], out_hbm.at[pl.ds((c * 2 + 1) * NPAD + s * RPT, RPT)])


@functools.cache
def _make_deg_kernel():
    mesh = plsc.VectorSubcoreMesh(core_axis_name="c", subcore_axis_name="s",
                                  num_cores=2, num_subcores=NT)
    return pl.kernel(
        _deg_body,
        out_type=jax.ShapeDtypeStruct((4 * NPAD,), jnp.float32),
        mesh=mesh,
        scratch_types=[
            [pltpu.VMEM((CHD,), jnp.int32) for _ in range(_DRING)],
            [pltpu.VMEM((CHD,), jnp.int32) for _ in range(_DRING)],
            pltpu.VMEM((CHD,), jnp.float32),
            pltpu.VMEM_SHARED((NPAD,), jnp.float32),
            pltpu.VMEM_SHARED((NPAD,), jnp.float32),
            [pltpu.SemaphoreType.DMA for _ in range(_DRING)],
            [pltpu.SemaphoreType.DMA for _ in range(_DRING)],
        ],
    )


def _deg_kernel(g_flat, zeros1, ones1):
    return _make_deg_kernel()(g_flat, zeros1, ones1)


_RING = 5
EDG_T1 = E // (2 * NT)
NIT1 = EDG_T1 // CH


def _make_segsum_body(goff):
    def _segsum_body(xs_hbm, g_hbm, zeros2_hbm, out_hbm, sidx, didx, rows,
                     acc, isem, gsem, ssem):
        c = lax.axis_index("c")
        s = lax.axis_index("s")
        pltpu.sync_copy(zeros2_hbm.at[pl.ds(s * RPT, RPT)], acc.at[pl.ds(s * RPT, RPT)])
        plsc.subcore_barrier()

        w = c * NT + s
        sbase = goff * (2 * E) + w * EDG_T1
        dbase = sbase + E

        def idx_start(t, b):
            pltpu.async_copy(g_hbm.at[pl.ds(sbase + t * CH, CH)], sidx[b], isem[b])
            pltpu.async_copy(g_hbm.at[pl.ds(dbase + t * CH, CH)], didx[b], isem[b])

        def idx_wait(t, b):
            pltpu.make_async_copy(g_hbm.at[pl.ds(sbase + t * CH, CH)], sidx[b], isem[b]).wait()
            pltpu.make_async_copy(g_hbm.at[pl.ds(dbase + t * CH, CH)], didx[b], isem[b]).wait()

        def gather_start(b):
            pltpu.async_copy(xs_hbm.at[sidx[b]], rows[b], gsem[b])

        def gather_wait(b):
            pltpu.make_async_copy(xs_hbm.at[sidx[b]], rows[b], gsem[b]).wait()

        def scatter_start(b):
            pltpu.async_copy(rows[b], acc.at[didx[b]], ssem[b], add=True)

        def scatter_wait(b):
            pltpu.make_async_copy(rows[b], acc.at[didx[b]], ssem[b]).wait()

        idx_start(0, 0)
        idx_start(1, 1)
        idx_wait(0, 0)
        gather_start(0)

        def body(g, carry):
            for b in range(_RING):
                t = _RING * g + b
                bi = (b + 2) % _RING

                @pl.when((t >= 3) & (t + 2 <= NIT1 - 1))
                def _():
                    scatter_wait(bi)

                @pl.when(t + 2 <= NIT1 - 1)
                def _():
                    idx_start(t + 2, bi)

                bg = (b + 1) % _RING

                @pl.when(t + 1 <= NIT1 - 1)
                def _():
                    idx_wait(t + 1, bg)
                    gather_start(bg)

                gather_wait(b)
                scatter_start(b)
            return carry

        lax.fori_loop(0, NIT1 // _RING, body, 0)
        for t in range(NIT1 - _RING, NIT1):
            scatter_wait(t % _RING)
        plsc.subcore_barrier()
        sl = pl.ds(s * RPT, RPT)
        pltpu.sync_copy(acc.at[sl], out_hbm.at[c, sl])

    return _segsum_body


@functools.cache
def _make_segsum_kernel(goff):
    mesh = plsc.VectorSubcoreMesh(core_axis_name="c", subcore_axis_name="s",
                                  num_cores=2, num_subcores=NT)
    return pl.kernel(
        _make_segsum_body(goff),
        out_type=jax.ShapeDtypeStruct((2, NPAD, D), jnp.float32),
        mesh=mesh,
        scratch_types=[
            [pltpu.VMEM((CH,), jnp.int32) for _ in range(_RING)],
            [pltpu.VMEM((CH,), jnp.int32) for _ in range(_RING)],
            [pltpu.VMEM((CH, D), jnp.float32) for _ in range(_RING)],
            pltpu.VMEM_SHARED((NPAD, D), jnp.float32),
            [pltpu.SemaphoreType.DMA for _ in range(_RING)],
            [pltpu.SemaphoreType.DMA for _ in range(_RING)],
            [pltpu.SemaphoreType.DMA for _ in range(_RING)],
        ],
    )


def _segsum_kernel(xs_g, g_flat, zeros2, goff):
    return _make_segsum_kernel(goff)(xs_g, g_flat, zeros2)


def _prescale_body(feat_ref, deg_ref, xs_ref, ns_ref, nd_ref):
    f = feat_ref[0]
    dsrc = deg_ref[0, 0]
    ddst = deg_ref[0, 1]
    ns = lax.rsqrt(jnp.maximum(dsrc, 1.0))
    nd = lax.rsqrt(jnp.maximum(ddst, 1.0))
    ns_ref[0] = ns
    nd_ref[0] = nd
    xs_ref[0] = f * ns


def _prescale_call(feats, degs4):
    rb = 1024
    nb = NPAD // rb
    return pl.pallas_call(
        _prescale_body,
        grid=(2, nb),
        in_specs=[
            pl.BlockSpec((1, rb, D), lambda g, i: (g, i, 0)),
            pl.BlockSpec((1, 2, rb, 1), lambda g, i: (g, 0, i, 0)),
        ],
        out_specs=[
            pl.BlockSpec((1, rb, D), lambda g, i: (g, i, 0)),
            pl.BlockSpec((1, rb, 1), lambda g, i: (g, i, 0)),
            pl.BlockSpec((1, rb, 1), lambda g, i: (g, i, 0)),
        ],
        out_shape=[
            jax.ShapeDtypeStruct((2, NPAD, D), jnp.float32),
            jax.ShapeDtypeStruct((2, NPAD, 1), jnp.float32),
            jax.ShapeDtypeStruct((2, NPAD, 1), jnp.float32),
        ],
    )(feats, degs4)


_RB = 512
_NB = NPAD // _RB


def _mlp1_body(agg_ref, ns_ref, nd_ref, w1_ref, b1_ref, w2_ref, ys_ref):
    a = agg_ref[0] + agg_ref[1]
    h = jnp.dot(a, w1_ref[...], preferred_element_type=jnp.float32)
    h = jnp.maximum(h * nd_ref[...] + b1_ref[...], 0.0)
    y = jnp.dot(h, w2_ref[...], preferred_element_type=jnp.float32)
    ys_ref[...] = y * ns_ref[...]


def _mlp1_call(agg, ns_g, nd_g, W1, b1, W2):
    return pl.pallas_call(
        _mlp1_body,
        grid=(_NB,),
        in_specs=[
            pl.BlockSpec((2, _RB, D), lambda i: (0, i, 0)),
            pl.BlockSpec((_RB, 1), lambda i: (i, 0)),
            pl.BlockSpec((_RB, 1), lambda i: (i, 0)),
            pl.BlockSpec((D, H2), lambda i: (0, 0)),
            pl.BlockSpec((1, H2), lambda i: (0, 0)),
            pl.BlockSpec((H2, D), lambda i: (0, 0)),
        ],
        out_specs=pl.BlockSpec((_RB, D), lambda i: (i, 0)),
        out_shape=jax.ShapeDtypeStruct((NPAD, D), jnp.float32),
    )(agg, ns_g, nd_g, W1, b1, W2)


def _proj_body(agg_ref, nd_ref, b2_ref, f1w_ref, f1b_ref, f2w_ref, f2b_ref,
               zn_ref):
    i = pl.program_id(0)
    a = agg_ref[0] + agg_ref[1]
    h = jnp.maximum(a * nd_ref[...] + b2_ref[...], 0.0)
    t = jnp.dot(h, f1w_ref[...], preferred_element_type=jnp.float32) + f1b_ref[...]
    e = jnp.where(t > 0.0, t, jnp.exp(t) - 1.0)
    z = jnp.dot(e, f2w_ref[...], preferred_element_type=jnp.float32) + f2b_ref[...]
    nrm = jnp.sqrt(jnp.sum(z * z, axis=1, keepdims=True))
    zn = z / jnp.maximum(nrm, 1e-12)
    rows = lax.broadcasted_iota(jnp.int32, (_RB, 1), 0) + i * _RB
    zn = jnp.where(rows < N, zn, 0.0)
    zn_ref[...] = zn.astype(jnp.bfloat16)


def _proj_call(agg2, nd_g, b2, fc1_W, fc1_b, fc2_W, fc2_b):
    return pl.pallas_call(
        _proj_body,
        grid=(_NB,),
        in_specs=[
            pl.BlockSpec((2, _RB, D), lambda i: (0, i, 0)),
            pl.BlockSpec((_RB, 1), lambda i: (i, 0)),
            pl.BlockSpec((1, D), lambda i: (0, 0)),
            pl.BlockSpec((D, D), lambda i: (0, 0)),
            pl.BlockSpec((1, D), lambda i: (0, 0)),
            pl.BlockSpec((D, D), lambda i: (0, 0)),
            pl.BlockSpec((1, D), lambda i: (0, 0)),
        ],
        out_specs=pl.BlockSpec((_RB, D), lambda i: (i, 0)),
        out_shape=jax.ShapeDtypeStruct((NPAD, D), jnp.bfloat16),
    )(agg2, nd_g, b2, fc1_W, fc1_b, fc2_W, fc2_b)


_RI = 512
_CJ = 2048
_NBI = NPAD // _RI
_NBJ = NPAD // _CJ
_DN = (((1,), (1,)), ((), ()))


def _lossA_body(zr_ref, zc_ref, out_ref, rsA):
    i = pl.program_id(0)
    j = pl.program_id(1)
    sa_m = lax.dot_general(zr_ref[...], zc_ref[...], _DN,
                           preferred_element_type=jnp.float32)
    colg = lax.broadcasted_iota(jnp.int32, (_RI, _CJ), 1) + j * _CJ
    zero = jnp.float32(0.0)
    ea = jnp.where(colg < N, jnp.exp(INV_T * sa_m), zero)
    sa = jnp.sum(ea, axis=1, keepdims=True)
    sl = pl.ds(i * _RI, _RI)

    @pl.when(j == 0)
    def _():
        rsA[sl, :] = sa

    @pl.when(j > 0)
    def _():
        rsA[sl, :] += sa

    @pl.when((i == _NBI - 1) & (j == _NBJ - 1))
    def _():
        out_ref[...] = rsA[...]


def _lossA_call(zn1):
    return pl.pallas_call(
        _lossA_body,
        grid=(_NBI, _NBJ),
        in_specs=[
            pl.BlockSpec((_RI, D), lambda i, j: (i, 0)),
            pl.BlockSpec((_CJ, D), lambda i, j: (j, 0)),
        ],
        out_specs=pl.BlockSpec((NPAD, 1), lambda i, j: (0, 0)),
        out_shape=jax.ShapeDtypeStruct((NPAD, 1), jnp.float32),
        scratch_shapes=[pltpu.VMEM((NPAD, 1), jnp.float32)],
    )(zn1, zn1)


def _lossB_body(z1r_ref, z2r_ref, z2c_ref, rsA_ref, out_ref, rsB, rsC, csC,
                dg):
    i = pl.program_id(0)
    j = pl.program_id(1)
    z1r = z1r_ref[...]
    z2r = z2r_ref[...]
    z2c = z2c_ref[...]
    sb_m = lax.dot_general(z2r, z2c, _DN, preferred_element_type=jnp.float32)
    sc_m = lax.dot_general(z1r, z2c, _DN, preferred_element_type=jnp.float32)
    colg = lax.broadcasted_iota(jnp.int32, (_RI, _CJ), 1) + j * _CJ
    rowg = lax.broadcasted_iota(jnp.int32, (_RI, _CJ), 0) + i * _RI
    cval = colg < N
    zero = jnp.float32(0.0)
    eb = jnp.where(cval, jnp.exp(INV_T * sb_m), zero)
    ec = jnp.where(cval & (rowg < N), jnp.exp(INV_T * sc_m), zero)
    sb = jnp.sum(eb, axis=1, keepdims=True)
    sc = jnp.sum(ec, axis=1, keepdims=True)
    cs = jnp.sum(ec, axis=0, keepdims=True)
    dd = jnp.sum(jnp.where(rowg == colg, INV_T * sc_m, zero), axis=1,
                 keepdims=True)
    sl = pl.ds(i * _RI, _RI)

    @pl.when(j == 0)
    def _():
        rsB[sl, :] = sb
        rsC[sl, :] = sc
        dg[sl, :] = dd

    @pl.when(j > 0)
    def _():
        rsB[sl, :] += sb
        rsC[sl, :] += sc
        dg[sl, :] += dd

    @pl.when(i == 0)
    def _():
        csC[j, :, :] = cs

    @pl.when(i > 0)
    def _():
        csC[j, :, :] += cs

    @pl.when((i == _NBI - 1) & (j == _NBJ - 1))
    def _():
        x1 = rsA_ref[...] + rsC[...] - E2
        lv1 = -dg[...] + 0.5 * jnp.log(x1)
        rows = lax.broadcasted_iota(jnp.int32, (NPAD, 1), 0)
        tot = jnp.sum(jnp.where(rows < N, lv1, zero), keepdims=True)
        rows_j = lax.broadcasted_iota(jnp.int32, (_CJ, 1), 0)
        for jj in range(_NBJ):
            col = jnp.transpose(csC[jj, :, :])
            x2 = rsB[pl.ds(jj * _CJ, _CJ), :] + col - E2
            lv2 = 0.5 * jnp.log(x2)
            val = (rows_j + jj * _CJ) < N
            tot = tot + jnp.sum(jnp.where(val, lv2, zero), keepdims=True)
        out_ref[...] = tot / N


def _lossB_call(zn1, zn2, rsA):
    return pl.pallas_call(
        _lossB_body,
        grid=(_NBI, _NBJ),
        in_specs=[
            pl.BlockSpec((_RI, D), lambda i, j: (i, 0)),
            pl.BlockSpec((_RI, D), lambda i, j: (i, 0)),
            pl.BlockSpec((_CJ, D), lambda i, j: (j, 0)),
            pl.BlockSpec((NPAD, 1), lambda i, j: (0, 0)),
        ],
        out_specs=pl.BlockSpec((1, 1), lambda i, j: (0, 0)),
        out_shape=jax.ShapeDtypeStruct((1, 1), jnp.float32),
        scratch_shapes=[
            pltpu.VMEM((NPAD, 1), jnp.float32),
            pltpu.VMEM((NPAD, 1), jnp.float32),
            pltpu.VMEM((_NBJ, 1, _CJ), jnp.float32),
            pltpu.VMEM((NPAD, 1), jnp.float32),
        ],
    )(zn1, zn2, zn2, rsA)


def kernel(graph1, graph2, feat1, feat2, W1, b1, W2, b2, fc1_W, fc1_b, fc2_W,
           fc2_b):
    g_flat = jnp.concatenate([graph1.astype(jnp.int32).reshape(-1),
                              graph2.astype(jnp.int32).reshape(-1)])
    f1p = jnp.pad(feat1, ((0, NPAD - N), (0, 0)))
    f2p = jnp.pad(feat2, ((0, NPAD - N), (0, 0)))
    feats = jnp.stack([f1p, f2p])
    zeros1 = jnp.zeros((NPAD,), jnp.float32)
    zeros2 = jnp.zeros((NPAD, D), jnp.float32)
    ones1 = jnp.ones((CHD,), jnp.float32)
    b1r = b1.reshape(1, H2)
    b2r = b2.reshape(1, D)
    f1br = fc1_b.reshape(1, D)
    f2br = fc2_b.reshape(1, D)

    degs = _deg_kernel(g_flat, zeros1, ones1)
    xs, ns, nd = _prescale_call(feats, degs.reshape(2, 2, NPAD, 1))
    a11 = _segsum_kernel(xs[0], g_flat, zeros2, 0)
    a12 = _segsum_kernel(xs[1], g_flat, zeros2, 1)
    ys1 = _mlp1_call(a11, ns[0], nd[0], W1, b1r, W2)
    ys2 = _mlp1_call(a12, ns[1], nd[1], W1, b1r, W2)
    a21 = _segsum_kernel(ys1, g_flat, zeros2, 0)
    a22 = _segsum_kernel(ys2, g_flat, zeros2, 1)
    zn1 = _proj_call(a21, nd[0], b2r, fc1_W, f1br, fc2_W, f2br)
    rsA = _lossA_call(zn1)
    zn2 = _proj_call(a22, nd[1], b2r, fc1_W, f1br, fc2_W, f2br)
    out = _lossB_call(zn1, zn2, rsA)
    return out.reshape(())


# speedup vs baseline: 10.9979x; 1.0289x over previous
import functools

import jax
import jax.numpy as jnp
import numpy as np
from jax import lax
from jax.experimental import pallas as pl
from jax.experimental.pallas import tpu as pltpu
import jax.experimental.pallas.tpu_sc as plsc

N = 10000
NPAD = 10240
E = 320000
D = 128
H2 = 256
TEMP = 0.5
INV_T = 1.0 / TEMP
E2 = float(np.exp(1.0 / TEMP))

NT = 16
EDG_T = E // NT
CH = 40
NIT = EDG_T // CH
RPT = NPAD // NT
EROWS = E // CH
G_ROWS = 4 * EROWS

CHD = 80
NITD = EDG_T // CHD
_DRING = 5


def _deg_body(g_hbm, zeros1_hbm, ones_hbm, out_hbm, sidx, didx, ones, acc_s,
              acc_d, isem, ssem):
    c = lax.axis_index("c")
    s = lax.axis_index("s")
    pltpu.sync_copy(ones_hbm, ones)
    pltpu.sync_copy(zeros1_hbm.at[pl.ds(s * RPT, RPT)], acc_s.at[pl.ds(s * RPT, RPT)])
    pltpu.sync_copy(zeros1_hbm.at[pl.ds(s * RPT, RPT)], acc_d.at[pl.ds(s * RPT, RPT)])
    plsc.subcore_barrier()

    sbase = c * (2 * E) + s * EDG_T
    dbase = sbase + E

    def idx_start(t, b):
        pltpu.async_copy(g_hbm.at[pl.ds(sbase + t * CHD, CHD)], sidx[b], isem[b])
        pltpu.async_copy(g_hbm.at[pl.ds(dbase + t * CHD, CHD)], didx[b], isem[b])

    def idx_wait(t, b):
        pltpu.make_async_copy(g_hbm.at[pl.ds(sbase + t * CHD, CHD)], sidx[b], isem[b]).wait()
        pltpu.make_async_copy(g_hbm.at[pl.ds(dbase + t * CHD, CHD)], didx[b], isem[b]).wait()

    def scatter_start(b):
        pltpu.async_copy(ones, acc_s.at[sidx[b]], ssem[b], add=True)
        pltpu.async_copy(ones, acc_d.at[didx[b]], ssem[b], add=True)

    def scatter_wait(b):
        pltpu.make_async_copy(ones, acc_s.at[sidx[b]], ssem[b]).wait()
        pltpu.make_async_copy(ones, acc_d.at[didx[b]], ssem[b]).wait()

    idx_start(0, 0)
    idx_start(1, 1)

    def body(g, carry):
        for b in range(_DRING):
            t = _DRING * g + b
            bi = (b + 2) % _DRING

            @pl.when((t >= 3) & (t + 2 <= NITD - 1))
            def _():
                scatter_wait(bi)

            @pl.when(t + 2 <= NITD - 1)
            def _():
                idx_start(t + 2, bi)

            idx_wait(t, b)
            scatter_start(b)
        return carry

    lax.fori_loop(0, NITD // _DRING, body, 0)
    for t in range(NITD - _DRING, NITD):
        scatter_wait(t % _DRING)
    plsc.subcore_barrier()
    sl = pl.ds(s * RPT, RPT)
    pltpu.sync_copy(acc_s.at[sl], out_hbm.at[pl.ds(c * 2 * NPAD + s * RPT, RPT)])
    pltpu.sync_copy(acc_d.at[sl], out_hbm.at[pl.ds((c * 2 + 1) * NPAD + s * RPT, RPT)])


@functools.cache
def _make_deg_kernel():
    mesh = plsc.VectorSubcoreMesh(core_axis_name="c", subcore_axis_name="s",
                                  num_cores=2, num_subcores=NT)
    return pl.kernel(
        _deg_body,
        out_type=jax.ShapeDtypeStruct((4 * NPAD,), jnp.float32),
        mesh=mesh,
        scratch_types=[
            [pltpu.VMEM((CHD,), jnp.int32) for _ in range(_DRING)],
            [pltpu.VMEM((CHD,), jnp.int32) for _ in range(_DRING)],
            pltpu.VMEM((CHD,), jnp.float32),
            pltpu.VMEM_SHARED((NPAD,), jnp.float32),
            pltpu.VMEM_SHARED((NPAD,), jnp.float32),
            [pltpu.SemaphoreType.DMA for _ in range(_DRING)],
            [pltpu.SemaphoreType.DMA for _ in range(_DRING)],
        ],
    )


def _deg_kernel(g_flat, zeros1, ones1):
    return _make_deg_kernel()(g_flat, zeros1, ones1)


_RING = 5
EDG_T1 = E // (2 * NT)
NIT1 = EDG_T1 // CH


def _make_segsum_body(goff):
    def _segsum_body(xs_hbm, g_hbm, zeros2_hbm, out_hbm, sidx, didx, rows,
                     acc, isem, gsem, ssem):
        c = lax.axis_index("c")
        s = lax.axis_index("s")
        pltpu.sync_copy(zeros2_hbm.at[pl.ds(s * RPT, RPT)], acc.at[pl.ds(s * RPT, RPT)])
        plsc.subcore_barrier()

        w = c * NT + s
        sbase = goff * (2 * E) + w * EDG_T1
        dbase = sbase + E

        def idx_start(t, b):
            pltpu.async_copy(g_hbm.at[pl.ds(sbase + t * CH, CH)], sidx[b], isem[b])
            pltpu.async_copy(g_hbm.at[pl.ds(dbase + t * CH, CH)], didx[b], isem[b])

        def idx_wait(t, b):
            pltpu.make_async_copy(g_hbm.at[pl.ds(sbase + t * CH, CH)], sidx[b], isem[b]).wait()
            pltpu.make_async_copy(g_hbm.at[pl.ds(dbase + t * CH, CH)], didx[b], isem[b]).wait()

        def gather_start(b):
            pltpu.async_copy(xs_hbm.at[sidx[b]], rows[b], gsem[b])

        def gather_wait(b):
            pltpu.make_async_copy(xs_hbm.at[sidx[b]], rows[b], gsem[b]).wait()

        def scatter_start(b):
            pltpu.async_copy(rows[b], acc.at[didx[b]], ssem[b], add=True)

        def scatter_wait(b):
            pltpu.make_async_copy(rows[b], acc.at[didx[b]], ssem[b]).wait()

        idx_start(0, 0)
        idx_start(1, 1)
        idx_wait(0, 0)
        gather_start(0)

        def body(g, carry):
            for b in range(_RING):
                t = _RING * g + b
                bi = (b + 2) % _RING

                @pl.when((t >= 3) & (t + 2 <= NIT1 - 1))
                def _():
                    scatter_wait(bi)

                @pl.when(t + 2 <= NIT1 - 1)
                def _():
                    idx_start(t + 2, bi)

                bg = (b + 1) % _RING

                @pl.when(t + 1 <= NIT1 - 1)
                def _():
                    idx_wait(t + 1, bg)
                    gather_start(bg)

                gather_wait(b)
                scatter_start(b)
            return carry

        lax.fori_loop(0, NIT1 // _RING, body, 0)
        for t in range(NIT1 - _RING, NIT1):
            scatter_wait(t % _RING)
        plsc.subcore_barrier()
        sl = pl.ds(s * RPT, RPT)
        pltpu.sync_copy(acc.at[sl], out_hbm.at[c, sl])

    return _segsum_body


@functools.cache
def _make_segsum_kernel(goff):
    mesh = plsc.VectorSubcoreMesh(core_axis_name="c", subcore_axis_name="s",
                                  num_cores=2, num_subcores=NT)
    return pl.kernel(
        _make_segsum_body(goff),
        out_type=jax.ShapeDtypeStruct((2, NPAD, D), jnp.float32),
        mesh=mesh,
        scratch_types=[
            [pltpu.VMEM((CH,), jnp.int32) for _ in range(_RING)],
            [pltpu.VMEM((CH,), jnp.int32) for _ in range(_RING)],
            [pltpu.VMEM((CH, D), jnp.float32) for _ in range(_RING)],
            pltpu.VMEM_SHARED((NPAD, D), jnp.float32),
            [pltpu.SemaphoreType.DMA for _ in range(_RING)],
            [pltpu.SemaphoreType.DMA for _ in range(_RING)],
            [pltpu.SemaphoreType.DMA for _ in range(_RING)],
        ],
    )


def _segsum_kernel(xs_g, g_flat, zeros2, goff):
    return _make_segsum_kernel(goff)(xs_g, g_flat, zeros2)


def _prescale_body(feat_ref, deg_ref, xs_ref, ns_ref, nd_ref):
    f = feat_ref[...]
    dsrc = deg_ref[0]
    ddst = deg_ref[1]
    ns = lax.rsqrt(jnp.maximum(dsrc, 1.0))
    nd = lax.rsqrt(jnp.maximum(ddst, 1.0))
    ns_ref[...] = ns
    nd_ref[...] = nd
    xs_ref[...] = f * ns


def _prescale_call(featp, deg_g):
    rb = 1024
    nb = NPAD // rb
    return pl.pallas_call(
        _prescale_body,
        grid=(nb,),
        in_specs=[
            pl.BlockSpec((rb, D), lambda i: (i, 0)),
            pl.BlockSpec((2, rb, 1), lambda i: (0, i, 0)),
        ],
        out_specs=[
            pl.BlockSpec((rb, D), lambda i: (i, 0)),
            pl.BlockSpec((rb, 1), lambda i: (i, 0)),
            pl.BlockSpec((rb, 1), lambda i: (i, 0)),
        ],
        out_shape=[
            jax.ShapeDtypeStruct((NPAD, D), jnp.float32),
            jax.ShapeDtypeStruct((NPAD, 1), jnp.float32),
            jax.ShapeDtypeStruct((NPAD, 1), jnp.float32),
        ],
    )(featp, deg_g)


_RB = 512
_NB = NPAD // _RB


def _mlp1_body(agg_ref, ns_ref, nd_ref, w1_ref, b1_ref, w2_ref, ys_ref):
    a = agg_ref[0] + agg_ref[1]
    h = jnp.dot(a, w1_ref[...], preferred_element_type=jnp.float32)
    h = jnp.maximum(h * nd_ref[...] + b1_ref[...], 0.0)
    y = jnp.dot(h, w2_ref[...], preferred_element_type=jnp.float32)
    ys_ref[...] = y * ns_ref[...]


def _mlp1_call(agg, ns_g, nd_g, W1, b1, W2):
    return pl.pallas_call(
        _mlp1_body,
        grid=(_NB,),
        in_specs=[
            pl.BlockSpec((2, _RB, D), lambda i: (0, i, 0)),
            pl.BlockSpec((_RB, 1), lambda i: (i, 0)),
            pl.BlockSpec((_RB, 1), lambda i: (i, 0)),
            pl.BlockSpec((D, H2), lambda i: (0, 0)),
            pl.BlockSpec((1, H2), lambda i: (0, 0)),
            pl.BlockSpec((H2, D), lambda i: (0, 0)),
        ],
        out_specs=pl.BlockSpec((_RB, D), lambda i: (i, 0)),
        out_shape=jax.ShapeDtypeStruct((NPAD, D), jnp.float32),
    )(agg, ns_g, nd_g, W1, b1, W2)


def _proj_body(agg_ref, nd_ref, b2_ref, f1w_ref, f1b_ref, f2w_ref, f2b_ref,
               zn_ref):
    i = pl.program_id(0)
    a = agg_ref[0] + agg_ref[1]
    h = jnp.maximum(a * nd_ref[...] + b2_ref[...], 0.0)
    t = jnp.dot(h, f1w_ref[...], preferred_element_type=jnp.float32) + f1b_ref[...]
    e = jnp.where(t > 0.0, t, jnp.exp(t) - 1.0)
    z = jnp.dot(e, f2w_ref[...], preferred_element_type=jnp.float32) + f2b_ref[...]
    nrm = jnp.sqrt(jnp.sum(z * z, axis=1, keepdims=True))
    zn = z / jnp.maximum(nrm, 1e-12)
    rows = lax.broadcasted_iota(jnp.int32, (_RB, 1), 0) + i * _RB
    zn = jnp.where(rows < N, zn, 0.0)
    zn_ref[...] = zn.astype(jnp.bfloat16)


def _proj_call(agg2, nd_g, b2, fc1_W, fc1_b, fc2_W, fc2_b):
    return pl.pallas_call(
        _proj_body,
        grid=(_NB,),
        in_specs=[
            pl.BlockSpec((2, _RB, D), lambda i: (0, i, 0)),
            pl.BlockSpec((_RB, 1), lambda i: (i, 0)),
            pl.BlockSpec((1, D), lambda i: (0, 0)),
            pl.BlockSpec((D, D), lambda i: (0, 0)),
            pl.BlockSpec((1, D), lambda i: (0, 0)),
            pl.BlockSpec((D, D), lambda i: (0, 0)),
            pl.BlockSpec((1, D), lambda i: (0, 0)),
        ],
        out_specs=pl.BlockSpec((_RB, D), lambda i: (i, 0)),
        out_shape=jax.ShapeDtypeStruct((NPAD, D), jnp.bfloat16),
    )(agg2, nd_g, b2, fc1_W, fc1_b, fc2_W, fc2_b)


_RI = 512
_CJ = 2048
_NBI = NPAD // _RI
_NBJ = NPAD // _CJ
_DN = (((1,), (1,)), ((), ()))


def _lossA_body(zr_ref, zc_ref, out_ref, rsA):
    i = pl.program_id(0)
    j = pl.program_id(1)
    sa_m = lax.dot_general(zr_ref[...], zc_ref[...], _DN,
                           preferred_element_type=jnp.float32)
    colg = lax.broadcasted_iota(jnp.int32, (_RI, _CJ), 1) + j * _CJ
    zero = jnp.float32(0.0)
    ea = jnp.where(colg < N, jnp.exp(INV_T * sa_m), zero)
    sa = jnp.sum(ea, axis=1, keepdims=True)
    sl = pl.ds(i * _RI, _RI)

    @pl.when(j == 0)
    def _():
        rsA[sl, :] = sa

    @pl.when(j > 0)
    def _():
        rsA[sl, :] += sa

    @pl.when((i == _NBI - 1) & (j == _NBJ - 1))
    def _():
        out_ref[...] = rsA[...]


def _lossA_call(zn1):
    return pl.pallas_call(
        _lossA_body,
        grid=(_NBI, _NBJ),
        in_specs=[
            pl.BlockSpec((_RI, D), lambda i, j: (i, 0)),
            pl.BlockSpec((_CJ, D), lambda i, j: (j, 0)),
        ],
        out_specs=pl.BlockSpec((NPAD, 1), lambda i, j: (0, 0)),
        out_shape=jax.ShapeDtypeStruct((NPAD, 1), jnp.float32),
        scratch_shapes=[pltpu.VMEM((NPAD, 1), jnp.float32)],
    )(zn1, zn1)


def _lossB_body(z1r_ref, z2r_ref, z2c_ref, rsA_ref, out_ref, rsB, rsC, csC,
                dg):
    i = pl.program_id(0)
    j = pl.program_id(1)
    z1r = z1r_ref[...]
    z2r = z2r_ref[...]
    z2c = z2c_ref[...]
    sb_m = lax.dot_general(z2r, z2c, _DN, preferred_element_type=jnp.float32)
    sc_m = lax.dot_general(z1r, z2c, _DN, preferred_element_type=jnp.float32)
    colg = lax.broadcasted_iota(jnp.int32, (_RI, _CJ), 1) + j * _CJ
    rowg = lax.broadcasted_iota(jnp.int32, (_RI, _CJ), 0) + i * _RI
    cval = colg < N
    zero = jnp.float32(0.0)
    eb = jnp.where(cval, jnp.exp(INV_T * sb_m), zero)
    ec = jnp.where(cval & (rowg < N), jnp.exp(INV_T * sc_m), zero)
    sb = jnp.sum(eb, axis=1, keepdims=True)
    sc = jnp.sum(ec, axis=1, keepdims=True)
    cs = jnp.sum(ec, axis=0, keepdims=True)
    dd = jnp.sum(jnp.where(rowg == colg, INV_T * sc_m, zero), axis=1,
                 keepdims=True)
    sl = pl.ds(i * _RI, _RI)

    @pl.when(j == 0)
    def _():
        rsB[sl, :] = sb
        rsC[sl, :] = sc
        dg[sl, :] = dd

    @pl.when(j > 0)
    def _():
        rsB[sl, :] += sb
        rsC[sl, :] += sc
        dg[sl, :] += dd

    @pl.when(i == 0)
    def _():
        csC[j, :, :] = cs

    @pl.when(i > 0)
    def _():
        csC[j, :, :] += cs

    @pl.when((i == _NBI - 1) & (j == _NBJ - 1))
    def _():
        x1 = rsA_ref[...] + rsC[...] - E2
        lv1 = -dg[...] + 0.5 * jnp.log(x1)
        rows = lax.broadcasted_iota(jnp.int32, (NPAD, 1), 0)
        tot = jnp.sum(jnp.where(rows < N, lv1, zero), keepdims=True)
        rows_j = lax.broadcasted_iota(jnp.int32, (_CJ, 1), 0)
        for jj in range(_NBJ):
            col = jnp.transpose(csC[jj, :, :])
            x2 = rsB[pl.ds(jj * _CJ, _CJ), :] + col - E2
            lv2 = 0.5 * jnp.log(x2)
            val = (rows_j + jj * _CJ) < N
            tot = tot + jnp.sum(jnp.where(val, lv2, zero), keepdims=True)
        out_ref[...] = tot / N


def _lossB_call(zn1, zn2, rsA):
    return pl.pallas_call(
        _lossB_body,
        grid=(_NBI, _NBJ),
        in_specs=[
            pl.BlockSpec((_RI, D), lambda i, j: (i, 0)),
            pl.BlockSpec((_RI, D), lambda i, j: (i, 0)),
            pl.BlockSpec((_CJ, D), lambda i, j: (j, 0)),
            pl.BlockSpec((NPAD, 1), lambda i, j: (0, 0)),
        ],
        out_specs=pl.BlockSpec((1, 1), lambda i, j: (0, 0)),
        out_shape=jax.ShapeDtypeStruct((1, 1), jnp.float32),
        scratch_shapes=[
            pltpu.VMEM((NPAD, 1), jnp.float32),
            pltpu.VMEM((NPAD, 1), jnp.float32),
            pltpu.VMEM((_NBJ, 1, _CJ), jnp.float32),
            pltpu.VMEM((NPAD, 1), jnp.float32),
        ],
    )(zn1, zn2, zn2, rsA)


def kernel(graph1, graph2, feat1, feat2, W1, b1, W2, b2, fc1_W, fc1_b, fc2_W,
           fc2_b):
    g_flat = jnp.concatenate([graph1.astype(jnp.int32).reshape(-1),
                              graph2.astype(jnp.int32).reshape(-1)])
    f1p = jnp.pad(feat1, ((0, NPAD - N), (0, 0)))
    f2p = jnp.pad(feat2, ((0, NPAD - N), (0, 0)))
    zeros1 = jnp.zeros((NPAD,), jnp.float32)
    zeros2 = jnp.zeros((NPAD, D), jnp.float32)
    ones1 = jnp.ones((CHD,), jnp.float32)
    b1r = b1.reshape(1, H2)
    b2r = b2.reshape(1, D)
    f1br = fc1_b.reshape(1, D)
    f2br = fc2_b.reshape(1, D)

    degs = _deg_kernel(g_flat, zeros1, ones1)
    degs4 = degs.reshape(2, 2, NPAD, 1)
    xs1, ns1, nd1 = _prescale_call(f1p, degs4[0])
    a11 = _segsum_kernel(xs1, g_flat, zeros2, 0)
    xs2, ns2, nd2 = _prescale_call(f2p, degs4[1])
    a12 = _segsum_kernel(xs2, g_flat, zeros2, 1)
    ys1 = _mlp1_call(a11, ns1, nd1, W1, b1r, W2)
    ys2 = _mlp1_call(a12, ns2, nd2, W1, b1r, W2)
    a21 = _segsum_kernel(ys1, g_flat, zeros2, 0)
    a22 = _segsum_kernel(ys2, g_flat, zeros2, 1)
    zn1 = _proj_call(a21, nd1, b2r, fc1_W, f1br, fc2_W, f2br)
    rsA = _lossA_call(zn1)
    zn2 = _proj_call(a22, nd2, b2r, fc1_W, f1br, fc2_W, f2br)
    out = _lossB_call(zn1, zn2, rsA)
    return out.reshape(())
